# baseline (speedup 1.0000x reference)
import functools

import jax
import jax.numpy as jnp
from jax import lax
from jax.experimental import pallas as pl
from jax.experimental.pallas import tpu as pltpu
from jax.experimental.pallas import tpu_sc as plsc

N_PAD = 10240
D = 128
CHUNK = 64
CPT = 162
EPW = CPT * CHUNK
NCORE = 2
NSUB = 16
NW = NCORE * NSUB
ET_PAD = NW * EPW
RPS = N_PAD // NSUB
BR = 256

_mesh = plsc.VectorSubcoreMesh(core_axis_name="c", subcore_axis_name="s",
                               num_cores=NCORE, num_subcores=NSUB)


def _ea_stats_body(ea_ref, o_ref):
    blk = ea_ref[...]
    o_ref[0, :] = jnp.sum(blk, axis=0)
    o_ref[1, :] = jnp.max(blk, axis=0)
    o_ref[2, :] = jnp.min(blk, axis=0)
    o_ref[3:, :] = jnp.zeros((5, 128), jnp.float32)


def _ea_stats(ea2d):
    return pl.pallas_call(
        _ea_stats_body,
        out_shape=jax.ShapeDtypeStruct((8, 128), jnp.float32),
    )(ea2d)


def _prep_x_body(h_ref, ws_ref, wd_ref, as_ref, ad_ref,
                 xs_ref, ss_ref, sd_ref, mx_ref):
    i = pl.program_id(0)
    h = h_ref[...]
    xs = jnp.dot(h, ws_ref[...], preferred_element_type=jnp.float32)
    xd = jnp.dot(h, wd_ref[...], preferred_element_type=jnp.float32)
    ss = jnp.dot(xs, as_ref[...], preferred_element_type=jnp.float32)
    sd = jnp.dot(xd, ad_ref[...], preferred_element_type=jnp.float32)
    xs_ref[...] = xs
    ss_ref[...] = ss
    sd_ref[...] = sd

    @pl.when(i == 0)
    def _():
        mx_ref[...] = jnp.full((8, 128), -jnp.inf)

    mx_ref[0, :] = jnp.maximum(mx_ref[0, :], jnp.max(ss))
    mx_ref[1, :] = jnp.maximum(mx_ref[1, :], jnp.max(sd))


def _prep_p_body(op_ref, s_ref, b_ref, ws_ref, wd_ref, as_ref, ad_ref,
                 xs_ref, ss_ref, sd_ref, mx_ref):
    i = pl.program_id(0)
    sb = s_ref[...]
    r = (1.0 / (sb[0] + sb[1] + 1e-16)).reshape(BR, 1)
    h = jnp.maximum((op_ref[0] + op_ref[1]) * r + b_ref[...], 0.0)
    xs = jnp.dot(h, ws_ref[...], preferred_element_type=jnp.float32)
    xd = jnp.dot(h, wd_ref[...], preferred_element_type=jnp.float32)
    ss = jnp.dot(xs, as_ref[...], preferred_element_type=jnp.float32)
    sd = jnp.dot(xd, ad_ref[...], preferred_element_type=jnp.float32)
    xs_ref[...] = xs
    ss_ref[...] = ss
    sd_ref[...] = sd

    @pl.when(i == 0)
    def _():
        mx_ref[...] = jnp.full((8, 128), -jnp.inf)

    mx_ref[0, :] = jnp.maximum(mx_ref[0, :], jnp.max(ss))
    mx_ref[1, :] = jnp.maximum(mx_ref[1, :], jnp.max(sd))


_PREP_OUT = (
    jax.ShapeDtypeStruct((N_PAD, D), jnp.float32),
    jax.ShapeDtypeStruct((N_PAD, 1), jnp.float32),
    jax.ShapeDtypeStruct((N_PAD, 1), jnp.float32),
    jax.ShapeDtypeStruct((8, 128), jnp.float32),
)
_PREP_OUT_SPECS = (
    pl.BlockSpec((BR, D), lambda i: (i, 0)),
    pl.BlockSpec((BR, 1), lambda i: (i, 0)),
    pl.BlockSpec((BR, 1), lambda i: (i, 0)),
    pl.BlockSpec((8, 128), lambda i: (0, 0)),
)
_W_SPECS = [
    pl.BlockSpec((D, D), lambda i: (0, 0)),
    pl.BlockSpec((D, D), lambda i: (0, 0)),
    pl.BlockSpec((D, 1), lambda i: (0, 0)),
    pl.BlockSpec((D, 1), lambda i: (0, 0)),
]


def _prep_from_x(h, ws, wd, a_s, a_d):
    return pl.pallas_call(
        _prep_x_body,
        grid=(N_PAD // BR,),
        in_specs=[pl.BlockSpec((BR, D), lambda i: (i, 0))] + _W_SPECS,
        out_specs=_PREP_OUT_SPECS,
        out_shape=_PREP_OUT,
    )(h, ws, wd, a_s, a_d)


def _prep_from_partials(op, s2, b, ws, wd, a_s, a_d):
    return pl.pallas_call(
        _prep_p_body,
        grid=(N_PAD // BR,),
        in_specs=[pl.BlockSpec((NCORE, BR, D), lambda i: (0, i, 0)),
                  pl.BlockSpec((NCORE, BR), lambda i: (0, i)),
                  pl.BlockSpec((1, D), lambda i: (0, 0))] + _W_SPECS,
        out_specs=_PREP_OUT_SPECS,
        out_shape=_PREP_OUT,
    )(op, s2, b.reshape(1, D), ws, wd, a_s, a_d)


def _final_body(o_ref, s_ref, b_ref, out_ref):
    r = 1.0 / (s_ref[0:1, :] + s_ref[1:2, :] + 1e-16)
    out_ref[...] = (o_ref[0:1, :] + o_ref[1:2, :]) * r + b_ref[0, 0]


def _final(o2, s2, b3):
    return pl.pallas_call(
        _final_body,
        in_specs=[pl.BlockSpec((NCORE, N_PAD), lambda: (0, 0)),
                  pl.BlockSpec((NCORE, N_PAD), lambda: (0, 0)),
                  pl.BlockSpec(memory_space=pltpu.SMEM)],
        out_shape=jax.ShapeDtypeStruct((1, N_PAD), jnp.float32),
    )(o2, s2, b3.reshape(1, 1))


def _zero_vec_ref(ref, nwords):
    z = jnp.zeros((16,), jnp.float32)

    def body(i, _):
        ref[pl.ds(i * 16, 16)] = z
        return 0

    lax.fori_loop(0, nwords // 16, body, 0)


def _zero_rows_ref(ref, nrows):
    z = jnp.zeros((16,), jnp.float32)

    def body(i, _):
        for j in range(D // 16):
            ref[i, pl.ds(j * 16, 16)] = z
        return 0

    lax.fori_loop(0, nrows, body, 0)


def _pass1_body(src_hbm, dst_hbm, ea_hbm, ss_hbm, sd_hbm, cb_hbm,
                p_hbm, s_hbm,
                ss_v, sd_v, src_v, dst_v, ea_v, p_v, stage_v, cb_v,
                s_sh, ssem):
    c = lax.axis_index("c")
    s = lax.axis_index("s")
    wid = s * NCORE + c

    _zero_vec_ref(stage_v, RPS)
    pltpu.sync_copy(stage_v, s_sh.at[pl.ds(s * RPS, RPS)])
    plsc.subcore_barrier()

    pltpu.sync_copy(ss_hbm, ss_v)
    pltpu.sync_copy(sd_hbm, sd_v)
    pltpu.sync_copy(cb_hbm, cb_v)
    pltpu.sync_copy(src_hbm.at[wid], src_v)
    pltpu.sync_copy(dst_hbm.at[wid], dst_v)
    pltpu.sync_copy(ea_hbm.at[wid], ea_v)

    cv = cb_v[0, :]
    bv = cb_v[1, :]

    def chunk(k, _):
        for j in range(CHUNK // 16):
            col = j * 16
            sidx = src_v[k, pl.ds(col, 16)]
            didx = dst_v[k, pl.ds(col, 16)]
            av = ea_v[k, pl.ds(col, 16)]
            e = (plsc.load_gather(ss_v, [sidx])
                 + plsc.load_gather(sd_v, [didx])
                 + cv * av)
            e = jnp.where(e >= 0.0, e, 0.2 * e)
            p_v[k, pl.ds(col, 16)] = jnp.exp(e - bv)
        return 0

    lax.fori_loop(0, CPT, chunk, 0)

    def scat(k, _):
        pltpu.async_copy(p_v.at[k], s_sh.at[dst_v.at[k]], ssem, add=True)
        return 0

    lax.fori_loop(0, CPT, scat, 0)

    pltpu.sync_copy(p_v, p_hbm.at[wid])

    def drain(k, _):
        pltpu.make_async_copy(p_v.at[k], s_sh.at[dst_v.at[k]], ssem).wait()
        return 0

    lax.fori_loop(0, CPT, drain, 0)
    plsc.subcore_barrier()

    pltpu.sync_copy(s_sh.at[pl.ds(s * RPS, RPS)], stage_v)
    pltpu.sync_copy(stage_v, s_hbm.at[c, pl.ds(s * RPS, RPS)])


_pass1 = pl.kernel(
    _pass1_body,
    out_type=(jax.ShapeDtypeStruct((NW, CPT, CHUNK), jnp.float32),
              jax.ShapeDtypeStruct((NCORE, N_PAD), jnp.float32)),
    mesh=_mesh,
    compiler_params=pltpu.CompilerParams(needs_layout_passes=False),
    scratch_types=[
        pltpu.VMEM((N_PAD,), jnp.float32),
        pltpu.VMEM((N_PAD,), jnp.float32),
        pltpu.VMEM((CPT, CHUNK), jnp.int32),
        pltpu.VMEM((CPT, CHUNK), jnp.int32),
        pltpu.VMEM((CPT, CHUNK), jnp.float32),
        pltpu.VMEM((CPT, CHUNK), jnp.float32),
        pltpu.VMEM((RPS,), jnp.float32),
        pltpu.VMEM((2, 16), jnp.float32),
        pltpu.VMEM_SHARED((N_PAD,), jnp.float32),
        pltpu.SemaphoreType.DMA,
    ],
)


def _scale_rows(rbuf, p_stage, nrows):
    def sgrp(g, _):
        av16 = p_stage[pl.ds(g * 16, 16)]
        for l in range(16):
            avec = jnp.full((16,), av16[l], jnp.float32)
            row = g * 16 + l
            for j in range(D // 16):
                col = j * 16
                rbuf[row, pl.ds(col, 16)] = rbuf[row, pl.ds(col, 16)] * avec
        return 0

    lax.fori_loop(0, nrows // 16, sgrp, 0)


C2 = 128
P2 = ET_PAD // NW // C2


def _pass2_body(src_hbm, dst_hbm, p_hbm, xs_hbm,
                o_hbm,
                src_st, dst_st, p_st, rows0_v, rows1_v,
                o_sh, g0, g1, d0, d1, sc0, sc1):
    c = lax.axis_index("c")
    s = lax.axis_index("s")
    wid = s * NCORE + c
    row0 = s * RPS

    _zero_rows_ref(rows0_v, C2)
    for z in range(RPS // C2):
        pltpu.sync_copy(rows0_v, o_sh.at[pl.ds(row0 + z * C2, C2)])
    plsc.subcore_barrier()

    bufs = ((rows0_v, g0, d0, sc0), (rows1_v, g1, d1, sc1))

    def issue_loads(k, slot, dsem):
        pltpu.async_copy(src_hbm.at[wid, k], src_st.at[slot], dsem)
        pltpu.async_copy(dst_hbm.at[wid, k], dst_st.at[slot], dsem)
        pltpu.async_copy(p_hbm.at[wid, k], p_st.at[slot], dsem)

    def wait_loads(k, slot, dsem):
        pltpu.make_async_copy(src_hbm.at[wid, k], src_st.at[slot],
                              dsem).wait()
        pltpu.make_async_copy(dst_hbm.at[wid, k], dst_st.at[slot],
                              dsem).wait()
        pltpu.make_async_copy(p_hbm.at[wid, k], p_st.at[slot], dsem).wait()

    issue_loads(0, 0, d0)
    wait_loads(0, 0, d0)
    pltpu.async_copy(xs_hbm.at[src_st.at[0]], rows0_v, g0)
    issue_loads(1, 1, d1)
    pltpu.make_async_copy(xs_hbm.at[src_st.at[0]], rows0_v, g0).wait()
    _scale_rows(rows0_v, p_st.at[0], C2)
    pltpu.async_copy(rows0_v, o_sh.at[dst_st.at[0]], sc0, add=True)
    wait_loads(1, 1, d1)
    pltpu.async_copy(xs_hbm.at[src_st.at[1]], rows1_v, g1)

    def step(k, par):
        rbuf, gsem, dsem, ssem = bufs[par]
        nbuf, ngsem, ndsem, nssem = bufs[1 - par]
        k1 = k + 1
        pltpu.make_async_copy(nbuf, o_sh.at[dst_st.at[1 - par]],
                              nssem).wait()

        @pl.when(k1 < P2)
        def _():
            issue_loads(k1, 1 - par, ndsem)

        pltpu.make_async_copy(xs_hbm.at[src_st.at[par]], rbuf, gsem).wait()
        _scale_rows(rbuf, p_st.at[par], C2)
        pltpu.async_copy(rbuf, o_sh.at[dst_st.at[par]], ssem, add=True)

        @pl.when(k1 < P2)
        def _():
            wait_loads(k1, 1 - par, ndsem)
            pltpu.async_copy(xs_hbm.at[src_st.at[1 - par]], nbuf, ngsem)

    def body(i, _):
        step(2 * i + 1, 1)
        step(2 * i + 2, 0)
        return 0

    lax.fori_loop(0, (P2 - 1) // 2, body, 0)
    pltpu.make_async_copy(rows0_v, o_sh.at[dst_st.at[0]], sc0).wait()
    plsc.subcore_barrier()

    for z in range(RPS // C2):
        pltpu.sync_copy(o_sh.at[pl.ds(row0 + z * C2, C2)], rows0_v)
        pltpu.sync_copy(rows0_v, o_hbm.at[c, pl.ds(row0 + z * C2, C2)])


_pass2 = pl.kernel(
    _pass2_body,
    out_type=jax.ShapeDtypeStruct((NCORE, N_PAD, D), jnp.float32),
    mesh=_mesh,
    compiler_params=pltpu.CompilerParams(needs_layout_passes=False),
    scratch_types=[
        pltpu.VMEM((2, C2), jnp.int32),
        pltpu.VMEM((2, C2), jnp.int32),
        pltpu.VMEM((2, C2), jnp.float32),
        pltpu.VMEM((C2, D), jnp.float32),
        pltpu.VMEM((C2, D), jnp.float32),
        pltpu.VMEM_SHARED((N_PAD, D), jnp.float32),
        pltpu.SemaphoreType.DMA,
        pltpu.SemaphoreType.DMA,
        pltpu.SemaphoreType.DMA,
        pltpu.SemaphoreType.DMA,
        pltpu.SemaphoreType.DMA,
        pltpu.SemaphoreType.DMA,
    ],
)


def _pass3_body(src_hbm, dst_hbm, ea_hbm, ss_hbm, sd_hbm, xs_hbm, cb_hbm,
                s_hbm, o_hbm,
                ss_v, sd_v, xs_v, src_v, dst_v, ea_v, p_v, v_v,
                stage_v, cb_v,
                s_sh, o_sh, psem, vsem):
    c = lax.axis_index("c")
    s = lax.axis_index("s")
    wid = s * NCORE + c

    _zero_vec_ref(stage_v, RPS)
    pltpu.sync_copy(stage_v, s_sh.at[pl.ds(s * RPS, RPS)])
    pltpu.sync_copy(stage_v, o_sh.at[pl.ds(s * RPS, RPS)])
    plsc.subcore_barrier()

    pltpu.sync_copy(ss_hbm, ss_v)
    pltpu.sync_copy(sd_hbm, sd_v)
    pltpu.sync_copy(xs_hbm, xs_v)
    pltpu.sync_copy(cb_hbm, cb_v)
    pltpu.sync_copy(src_hbm.at[wid], src_v)
    pltpu.sync_copy(dst_hbm.at[wid], dst_v)
    pltpu.sync_copy(ea_hbm.at[wid], ea_v)

    cv = cb_v[0, :]
    bv = cb_v[1, :]

    def chunk(k, _):
        for j in range(CHUNK // 16):
            col = k * CHUNK + j * 16
            sidx = src_v[pl.ds(col, 16)]
            didx = dst_v[k, pl.ds(j * 16, 16)]
            av = ea_v[pl.ds(col, 16)]
            e = (plsc.load_gather(ss_v, [sidx])
                 + plsc.load_gather(sd_v, [didx])
                 + cv * av)
            e = jnp.where(e >= 0.0, e, 0.2 * e)
            p = jnp.exp(e - bv)
            p_v[pl.ds(col, 16)] = p
            v_v[pl.ds(col, 16)] = p * plsc.load_gather(xs_v, [sidx])
        return 0

    lax.fori_loop(0, CPT, chunk, 0)

    def scat(k, _):
        pltpu.async_copy(p_v.at[pl.ds(k * CHUNK, CHUNK)],
                         s_sh.at[dst_v.at[k]], psem, add=True)
        pltpu.async_copy(v_v.at[pl.ds(k * CHUNK, CHUNK)],
                         o_sh.at[dst_v.at[k]], vsem, add=True)
        return 0

    lax.fori_loop(0, CPT, scat, 0)

    def drain(k, _):
        pltpu.make_async_copy(p_v.at[pl.ds(k * CHUNK, CHUNK)],
                              s_sh.at[dst_v.at[k]], psem).wait()
        pltpu.make_async_copy(v_v.at[pl.ds(k * CHUNK, CHUNK)],
                              o_sh.at[dst_v.at[k]], vsem).wait()
        return 0

    lax.fori_loop(0, CPT, drain, 0)
    plsc.subcore_barrier()

    pltpu.sync_copy(s_sh.at[pl.ds(s * RPS, RPS)], stage_v)
    pltpu.sync_copy(stage_v, s_hbm.at[c, pl.ds(s * RPS, RPS)])
    pltpu.sync_copy(o_sh.at[pl.ds(s * RPS, RPS)], stage_v)
    pltpu.sync_copy(stage_v, o_hbm.at[c, pl.ds(s * RPS, RPS)])


_pass3 = pl.kernel(
    _pass3_body,
    out_type=(jax.ShapeDtypeStruct((NCORE, N_PAD), jnp.float32),
              jax.ShapeDtypeStruct((NCORE, N_PAD), jnp.float32)),
    mesh=_mesh,
    compiler_params=pltpu.CompilerParams(needs_layout_passes=False),
    scratch_types=[
        pltpu.VMEM((N_PAD,), jnp.float32),
        pltpu.VMEM((N_PAD,), jnp.float32),
        pltpu.VMEM((N_PAD,), jnp.float32),
        pltpu.VMEM((EPW,), jnp.int32),
        pltpu.VMEM((CPT, CHUNK), jnp.int32),
        pltpu.VMEM((EPW,), jnp.float32),
        pltpu.VMEM((EPW,), jnp.float32),
        pltpu.VMEM((EPW,), jnp.float32),
        pltpu.VMEM((RPS,), jnp.float32),
        pltpu.VMEM((2, 16), jnp.float32),
        pltpu.VMEM_SHARED((N_PAD,), jnp.float32),
        pltpu.VMEM_SHARED((N_PAD,), jnp.float32),
        pltpu.SemaphoreType.DMA,
        pltpu.SemaphoreType.DMA,
    ],
)


def kernel(x, edge_index, edge_attr,
           W1s, W1d, W1e, a1s, a1d, a1e, b1,
           W2s, W2d, W2e, a2s, a2d, a2e, b2,
           W3s, W3d, W3e, a3s, a3d, a3e, b3):
    n = x.shape[0]
    e = edge_index.shape[1]
    pad_n = ET_PAD - e - n

    x_pad = jnp.zeros((N_PAD, D), jnp.float32).at[:n].set(x)
    loop = jnp.arange(n, dtype=jnp.int32)
    pad_idx = (n + (jnp.arange(pad_n, dtype=jnp.int32) % (N_PAD - n)))
    src = jnp.concatenate([edge_index[0].astype(jnp.int32), loop, pad_idx])
    dst = jnp.concatenate([edge_index[1].astype(jnp.int32), loop, pad_idx])
    src3 = src.reshape(NW, CPT, CHUNK)
    dst3 = dst.reshape(NW, CPT, CHUNK)
    src2f = src.reshape(NW, EPW)

    st = _ea_stats(edge_attr.reshape(-1, 128))
    ea_mean = jnp.sum(st[0, :]) / e
    ea_max = jnp.max(st[1, :])
    ea_min = jnp.min(st[2, :])
    ea_full = jnp.concatenate([
        edge_attr.reshape(-1), jnp.full((n,), ea_mean, jnp.float32),
        jnp.zeros((pad_n,), jnp.float32)]).reshape(NW, CPT, CHUNK)

    W3s_p = jnp.zeros((D, D), jnp.float32).at[:, :1].set(W3s)
    W3d_p = jnp.zeros((D, D), jnp.float32).at[:, :1].set(W3d)
    a3s_p = jnp.zeros((D, 1), jnp.float32).at[:1].set(a3s[:, None])
    a3d_p = jnp.zeros((D, 1), jnp.float32).at[:1].set(a3d[:, None])

    layers = [
        (W1s, W1d, a1s.reshape(D, 1), a1d.reshape(D, 1), W1e, a1e, b1),
        (W2s, W2d, a2s.reshape(D, 1), a2d.reshape(D, 1), W2e, a2e, b2),
        (W3s_p, W3d_p, a3s_p, a3d_p, W3e, a3e, b3),
    ]

    op = None
    r = None
    for li, (ws, wd, avs, avd, we, ave, b) in enumerate(layers):
        if li == 0:
            xs, ss, sd, mx = _prep_from_x(x_pad, ws, wd, avs, avd)
        else:
            prev_b = layers[li - 1][6]
            xs, ss, sd, mx = _prep_from_partials(
                op, s2, prev_b, ws, wd, avs, avd)
        cl = jnp.sum(we[0] * ave)
        se_max = jnp.maximum(cl * ea_max, cl * ea_min)
        bb = mx[0, 0] + mx[1, 0] + se_max
        bb = jnp.where(bb >= 0.0, bb, 0.2 * bb)
        cb = jnp.stack([jnp.full((16,), cl, jnp.float32),
                        jnp.full((16,), bb, jnp.float32)])

        if li < 2:
            p, s2 = _pass1(src3, dst3, ea_full,
                           ss.reshape(N_PAD), sd.reshape(N_PAD), cb)
            op = _pass2(src.reshape(NW, P2, C2), dst.reshape(NW, P2, C2),
                        p.reshape(NW, P2, C2), xs)
        else:
            s2, o2 = _pass3(src2f, dst3, ea_full.reshape(NW, EPW),
                            ss.reshape(N_PAD), sd.reshape(N_PAD),
                            xs[:, 0], cb)
            out = _final(o2, s2, b)
    return out.reshape(N_PAD, 1)[:n]


# speedup vs baseline: 1.0014x; 1.0014x over previous
import functools

import jax
import jax.numpy as jnp
from jax import lax
from jax.experimental import pallas as pl
from jax.experimental.pallas import tpu as pltpu
from jax.experimental.pallas import tpu_sc as plsc

N_PAD = 10240
D = 128
CHUNK = 64
CPT = 162
EPW = CPT * CHUNK
NCORE = 2
NSUB = 16
NW = NCORE * NSUB
ET_PAD = NW * EPW
RPS = N_PAD // NSUB
BR = 256

_mesh = plsc.VectorSubcoreMesh(core_axis_name="c", subcore_axis_name="s",
                               num_cores=NCORE, num_subcores=NSUB)


def _ea_stats_body(ea_ref, o_ref):
    blk = ea_ref[...]
    o_ref[0, :] = jnp.sum(blk, axis=0)
    o_ref[1, :] = jnp.max(blk, axis=0)
    o_ref[2, :] = jnp.min(blk, axis=0)
    o_ref[3:, :] = jnp.zeros((5, 128), jnp.float32)


def _ea_stats(ea2d):
    return pl.pallas_call(
        _ea_stats_body,
        out_shape=jax.ShapeDtypeStruct((8, 128), jnp.float32),
    )(ea2d)


def _prep_x_body(h_ref, ws_ref, wd_ref, as_ref, ad_ref,
                 xs_ref, ss_ref, sd_ref, mx_ref):
    i = pl.program_id(0)
    h = h_ref[...]
    xs = jnp.dot(h, ws_ref[...], preferred_element_type=jnp.float32)
    xd = jnp.dot(h, wd_ref[...], preferred_element_type=jnp.float32)
    ss = jnp.dot(xs, as_ref[...], preferred_element_type=jnp.float32)
    sd = jnp.dot(xd, ad_ref[...], preferred_element_type=jnp.float32)
    xs_ref[...] = xs
    ss_ref[...] = ss
    sd_ref[...] = sd

    @pl.when(i == 0)
    def _():
        mx_ref[...] = jnp.full((8, 128), -jnp.inf)

    mx_ref[0, :] = jnp.maximum(mx_ref[0, :], jnp.max(ss))
    mx_ref[1, :] = jnp.maximum(mx_ref[1, :], jnp.max(sd))


def _prep_p_body(op_ref, s_ref, b_ref, ws_ref, wd_ref, as_ref, ad_ref,
                 xs_ref, ss_ref, sd_ref, mx_ref):
    i = pl.program_id(0)
    sb = s_ref[...]
    r = (1.0 / (sb[0] + sb[1] + 1e-16)).reshape(BR, 1)
    h = jnp.maximum((op_ref[0] + op_ref[1]) * r + b_ref[...], 0.0)
    xs = jnp.dot(h, ws_ref[...], preferred_element_type=jnp.float32)
    xd = jnp.dot(h, wd_ref[...], preferred_element_type=jnp.float32)
    ss = jnp.dot(xs, as_ref[...], preferred_element_type=jnp.float32)
    sd = jnp.dot(xd, ad_ref[...], preferred_element_type=jnp.float32)
    xs_ref[...] = xs
    ss_ref[...] = ss
    sd_ref[...] = sd

    @pl.when(i == 0)
    def _():
        mx_ref[...] = jnp.full((8, 128), -jnp.inf)

    mx_ref[0, :] = jnp.maximum(mx_ref[0, :], jnp.max(ss))
    mx_ref[1, :] = jnp.maximum(mx_ref[1, :], jnp.max(sd))


_PREP_OUT = (
    jax.ShapeDtypeStruct((N_PAD, D), jnp.float32),
    jax.ShapeDtypeStruct((N_PAD, 1), jnp.float32),
    jax.ShapeDtypeStruct((N_PAD, 1), jnp.float32),
    jax.ShapeDtypeStruct((8, 128), jnp.float32),
)
_PREP_OUT_SPECS = (
    pl.BlockSpec((BR, D), lambda i: (i, 0)),
    pl.BlockSpec((BR, 1), lambda i: (i, 0)),
    pl.BlockSpec((BR, 1), lambda i: (i, 0)),
    pl.BlockSpec((8, 128), lambda i: (0, 0)),
)
_W_SPECS = [
    pl.BlockSpec((D, D), lambda i: (0, 0)),
    pl.BlockSpec((D, D), lambda i: (0, 0)),
    pl.BlockSpec((D, 1), lambda i: (0, 0)),
    pl.BlockSpec((D, 1), lambda i: (0, 0)),
]


def _prep_from_x(h, ws, wd, a_s, a_d):
    return pl.pallas_call(
        _prep_x_body,
        grid=(N_PAD // BR,),
        in_specs=[pl.BlockSpec((BR, D), lambda i: (i, 0))] + _W_SPECS,
        out_specs=_PREP_OUT_SPECS,
        out_shape=_PREP_OUT,
    )(h, ws, wd, a_s, a_d)


def _prep_from_partials(op, s2, b, ws, wd, a_s, a_d):
    return pl.pallas_call(
        _prep_p_body,
        grid=(N_PAD // BR,),
        in_specs=[pl.BlockSpec((NCORE, BR, D), lambda i: (0, i, 0)),
                  pl.BlockSpec((NCORE, BR), lambda i: (0, i)),
                  pl.BlockSpec((1, D), lambda i: (0, 0))] + _W_SPECS,
        out_specs=_PREP_OUT_SPECS,
        out_shape=_PREP_OUT,
    )(op, s2, b.reshape(1, D), ws, wd, a_s, a_d)


def _final_body(o_ref, s_ref, b_ref, out_ref):
    r = 1.0 / (s_ref[0:1, :] + s_ref[1:2, :] + 1e-16)
    out_ref[...] = (o_ref[0:1, :] + o_ref[1:2, :]) * r + b_ref[0, 0]


def _final(o2, s2, b3):
    return pl.pallas_call(
        _final_body,
        in_specs=[pl.BlockSpec((NCORE, N_PAD), lambda: (0, 0)),
                  pl.BlockSpec((NCORE, N_PAD), lambda: (0, 0)),
                  pl.BlockSpec(memory_space=pltpu.SMEM)],
        out_shape=jax.ShapeDtypeStruct((1, N_PAD), jnp.float32),
    )(o2, s2, b3.reshape(1, 1))


def _zero_vec_ref(ref, nwords):
    z = jnp.zeros((16,), jnp.float32)

    def body(i, _):
        ref[pl.ds(i * 16, 16)] = z
        return 0

    lax.fori_loop(0, nwords // 16, body, 0)


def _zero_rows_ref(ref, nrows):
    z = jnp.zeros((16,), jnp.float32)

    def body(i, _):
        for j in range(D // 16):
            ref[i, pl.ds(j * 16, 16)] = z
        return 0

    lax.fori_loop(0, nrows, body, 0)


def _pass1_body(src_hbm, dst_hbm, ea_hbm, ss_hbm, sd_hbm, cb_hbm,
                p_hbm, s_hbm,
                ss_v, sd_v, src_v, dst_v, ea_v, p_v, stage_v, cb_v,
                s_sh, ssem):
    c = lax.axis_index("c")
    s = lax.axis_index("s")
    wid = s * NCORE + c

    _zero_vec_ref(stage_v, RPS)
    pltpu.sync_copy(stage_v, s_sh.at[pl.ds(s * RPS, RPS)])
    plsc.subcore_barrier()

    pltpu.sync_copy(ss_hbm, ss_v)
    pltpu.sync_copy(sd_hbm, sd_v)
    pltpu.sync_copy(cb_hbm, cb_v)
    pltpu.sync_copy(src_hbm.at[wid], src_v)
    pltpu.sync_copy(dst_hbm.at[wid], dst_v)
    pltpu.sync_copy(ea_hbm.at[wid], ea_v)

    cv = cb_v[0, :]
    bv = cb_v[1, :]

    def chunk(k, _):
        for j in range(CHUNK // 16):
            col = j * 16
            sidx = src_v[k, pl.ds(col, 16)]
            didx = dst_v[k, pl.ds(col, 16)]
            av = ea_v[k, pl.ds(col, 16)]
            e = (plsc.load_gather(ss_v, [sidx])
                 + plsc.load_gather(sd_v, [didx])
                 + cv * av)
            e = jnp.where(e >= 0.0, e, 0.2 * e)
            p_v[k, pl.ds(col, 16)] = jnp.exp(e - bv)
        return 0

    lax.fori_loop(0, CPT, chunk, 0)

    def scat(k, _):
        pltpu.async_copy(p_v.at[k], s_sh.at[dst_v.at[k]], ssem, add=True)
        return 0

    lax.fori_loop(0, CPT, scat, 0)

    pltpu.sync_copy(p_v, p_hbm.at[wid])

    def drain(k, _):
        pltpu.make_async_copy(p_v.at[k], s_sh.at[dst_v.at[k]], ssem).wait()
        return 0

    lax.fori_loop(0, CPT, drain, 0)
    plsc.subcore_barrier()

    pltpu.sync_copy(s_sh.at[pl.ds(s * RPS, RPS)], stage_v)
    pltpu.sync_copy(stage_v, s_hbm.at[c, pl.ds(s * RPS, RPS)])


_pass1 = pl.kernel(
    _pass1_body,
    out_type=(jax.ShapeDtypeStruct((NW, CPT, CHUNK), jnp.float32),
              jax.ShapeDtypeStruct((NCORE, N_PAD), jnp.float32)),
    mesh=_mesh,
    compiler_params=pltpu.CompilerParams(needs_layout_passes=False),
    scratch_types=[
        pltpu.VMEM((N_PAD,), jnp.float32),
        pltpu.VMEM((N_PAD,), jnp.float32),
        pltpu.VMEM((CPT, CHUNK), jnp.int32),
        pltpu.VMEM((CPT, CHUNK), jnp.int32),
        pltpu.VMEM((CPT, CHUNK), jnp.float32),
        pltpu.VMEM((CPT, CHUNK), jnp.float32),
        pltpu.VMEM((RPS,), jnp.float32),
        pltpu.VMEM((2, 16), jnp.float32),
        pltpu.VMEM_SHARED((N_PAD,), jnp.float32),
        pltpu.SemaphoreType.DMA,
    ],
)


def _scale_rows(rbuf, p_stage, nrows):
    def sgrp(g, _):
        av16 = p_stage[pl.ds(g * 16, 16)]
        for l in range(16):
            avec = jnp.full((16,), av16[l], jnp.float32)
            row = g * 16 + l
            for j in range(D // 16):
                col = j * 16
                rbuf[row, pl.ds(col, 16)] = rbuf[row, pl.ds(col, 16)] * avec
        return 0

    lax.fori_loop(0, nrows // 16, sgrp, 0)


C2 = 128
P2 = ET_PAD // NW // C2


def _pass2_body(src_hbm, dst_hbm, p_hbm, xs_hbm,
                o_hbm,
                src_st, dst_st, p_st, rows0_v, rows1_v,
                o_sh, g0, g1, d0, d1, sc0, sc1):
    c = lax.axis_index("c")
    s = lax.axis_index("s")
    wid = s * NCORE + c
    row0 = s * RPS

    _zero_rows_ref(rows0_v, C2)
    for z in range(RPS // C2):
        pltpu.sync_copy(rows0_v, o_sh.at[pl.ds(row0 + z * C2, C2)])
    plsc.subcore_barrier()

    bufs = ((rows0_v, g0, d0, sc0), (rows1_v, g1, d1, sc1))

    def issue_loads(k, slot, dsem):
        pltpu.async_copy(src_hbm.at[wid, k], src_st.at[slot], dsem)
        pltpu.async_copy(dst_hbm.at[wid, k], dst_st.at[slot], dsem)
        pltpu.async_copy(p_hbm.at[wid, k], p_st.at[slot], dsem)

    def wait_loads(k, slot, dsem):
        pltpu.make_async_copy(src_hbm.at[wid, k], src_st.at[slot],
                              dsem).wait()
        pltpu.make_async_copy(dst_hbm.at[wid, k], dst_st.at[slot],
                              dsem).wait()
        pltpu.make_async_copy(p_hbm.at[wid, k], p_st.at[slot], dsem).wait()

    issue_loads(0, 0, d0)
    wait_loads(0, 0, d0)
    pltpu.async_copy(xs_hbm.at[src_st.at[0]], rows0_v, g0)
    issue_loads(1, 1, d1)
    pltpu.make_async_copy(xs_hbm.at[src_st.at[0]], rows0_v, g0).wait()
    _scale_rows(rows0_v, p_st.at[0], C2)
    pltpu.async_copy(rows0_v, o_sh.at[dst_st.at[0]], sc0, add=True)
    wait_loads(1, 1, d1)
    pltpu.async_copy(xs_hbm.at[src_st.at[1]], rows1_v, g1)

    def step(k, par):
        rbuf, gsem, dsem, ssem = bufs[par]
        nbuf, ngsem, ndsem, nssem = bufs[1 - par]
        k1 = k + 1
        pltpu.make_async_copy(nbuf, o_sh.at[dst_st.at[1 - par]],
                              nssem).wait()

        @pl.when(k1 < P2)
        def _():
            issue_loads(k1, 1 - par, ndsem)

        pltpu.make_async_copy(xs_hbm.at[src_st.at[par]], rbuf, gsem).wait()
        _scale_rows(rbuf, p_st.at[par], C2)

        @pl.when(k1 < P2)
        def _():
            wait_loads(k1, 1 - par, ndsem)
            pltpu.async_copy(xs_hbm.at[src_st.at[1 - par]], nbuf, ngsem)

        pltpu.async_copy(rbuf, o_sh.at[dst_st.at[par]], ssem, add=True)

    def body(i, _):
        step(2 * i + 1, 1)
        step(2 * i + 2, 0)
        return 0

    lax.fori_loop(0, (P2 - 1) // 2, body, 0)
    pltpu.make_async_copy(rows0_v, o_sh.at[dst_st.at[0]], sc0).wait()
    plsc.subcore_barrier()

    for z in range(RPS // C2):
        pltpu.sync_copy(o_sh.at[pl.ds(row0 + z * C2, C2)], rows0_v)
        pltpu.sync_copy(rows0_v, o_hbm.at[c, pl.ds(row0 + z * C2, C2)])


_pass2 = pl.kernel(
    _pass2_body,
    out_type=jax.ShapeDtypeStruct((NCORE, N_PAD, D), jnp.float32),
    mesh=_mesh,
    compiler_params=pltpu.CompilerParams(needs_layout_passes=False),
    scratch_types=[
        pltpu.VMEM((2, C2), jnp.int32),
        pltpu.VMEM((2, C2), jnp.int32),
        pltpu.VMEM((2, C2), jnp.float32),
        pltpu.VMEM((C2, D), jnp.float32),
        pltpu.VMEM((C2, D), jnp.float32),
        pltpu.VMEM_SHARED((N_PAD, D), jnp.float32),
        pltpu.SemaphoreType.DMA,
        pltpu.SemaphoreType.DMA,
        pltpu.SemaphoreType.DMA,
        pltpu.SemaphoreType.DMA,
        pltpu.SemaphoreType.DMA,
        pltpu.SemaphoreType.DMA,
    ],
)


def _pass3_body(src_hbm, dst_hbm, ea_hbm, ss_hbm, sd_hbm, xs_hbm, cb_hbm,
                s_hbm, o_hbm,
                ss_v, sd_v, xs_v, src_v, dst_v, ea_v, p_v, v_v,
                stage_v, cb_v,
                s_sh, o_sh, psem, vsem):
    c = lax.axis_index("c")
    s = lax.axis_index("s")
    wid = s * NCORE + c

    _zero_vec_ref(stage_v, RPS)
    pltpu.sync_copy(stage_v, s_sh.at[pl.ds(s * RPS, RPS)])
    pltpu.sync_copy(stage_v, o_sh.at[pl.ds(s * RPS, RPS)])
    plsc.subcore_barrier()

    pltpu.sync_copy(ss_hbm, ss_v)
    pltpu.sync_copy(sd_hbm, sd_v)
    pltpu.sync_copy(xs_hbm, xs_v)
    pltpu.sync_copy(cb_hbm, cb_v)
    pltpu.sync_copy(src_hbm.at[wid], src_v)
    pltpu.sync_copy(dst_hbm.at[wid], dst_v)
    pltpu.sync_copy(ea_hbm.at[wid], ea_v)

    cv = cb_v[0, :]
    bv = cb_v[1, :]

    def chunk(k, _):
        for j in range(CHUNK // 16):
            col = k * CHUNK + j * 16
            sidx = src_v[pl.ds(col, 16)]
            didx = dst_v[k, pl.ds(j * 16, 16)]
            av = ea_v[pl.ds(col, 16)]
            e = (plsc.load_gather(ss_v, [sidx])
                 + plsc.load_gather(sd_v, [didx])
                 + cv * av)
            e = jnp.where(e >= 0.0, e, 0.2 * e)
            p = jnp.exp(e - bv)
            p_v[pl.ds(col, 16)] = p
            v_v[pl.ds(col, 16)] = p * plsc.load_gather(xs_v, [sidx])
        return 0

    lax.fori_loop(0, CPT, chunk, 0)

    def scat(k, _):
        pltpu.async_copy(p_v.at[pl.ds(k * CHUNK, CHUNK)],
                         s_sh.at[dst_v.at[k]], psem, add=True)
        pltpu.async_copy(v_v.at[pl.ds(k * CHUNK, CHUNK)],
                         o_sh.at[dst_v.at[k]], vsem, add=True)
        return 0

    lax.fori_loop(0, CPT, scat, 0)

    def drain(k, _):
        pltpu.make_async_copy(p_v.at[pl.ds(k * CHUNK, CHUNK)],
                              s_sh.at[dst_v.at[k]], psem).wait()
        pltpu.make_async_copy(v_v.at[pl.ds(k * CHUNK, CHUNK)],
                              o_sh.at[dst_v.at[k]], vsem).wait()
        return 0

    lax.fori_loop(0, CPT, drain, 0)
    plsc.subcore_barrier()

    pltpu.sync_copy(s_sh.at[pl.ds(s * RPS, RPS)], stage_v)
    pltpu.sync_copy(stage_v, s_hbm.at[c, pl.ds(s * RPS, RPS)])
    pltpu.sync_copy(o_sh.at[pl.ds(s * RPS, RPS)], stage_v)
    pltpu.sync_copy(stage_v, o_hbm.at[c, pl.ds(s * RPS, RPS)])


_pass3 = pl.kernel(
    _pass3_body,
    out_type=(jax.ShapeDtypeStruct((NCORE, N_PAD), jnp.float32),
              jax.ShapeDtypeStruct((NCORE, N_PAD), jnp.float32)),
    mesh=_mesh,
    compiler_params=pltpu.CompilerParams(needs_layout_passes=False),
    scratch_types=[
        pltpu.VMEM((N_PAD,), jnp.float32),
        pltpu.VMEM((N_PAD,), jnp.float32),
        pltpu.VMEM((N_PAD,), jnp.float32),
        pltpu.VMEM((EPW,), jnp.int32),
        pltpu.VMEM((CPT, CHUNK), jnp.int32),
        pltpu.VMEM((EPW,), jnp.float32),
        pltpu.VMEM((EPW,), jnp.float32),
        pltpu.VMEM((EPW,), jnp.float32),
        pltpu.VMEM((RPS,), jnp.float32),
        pltpu.VMEM((2, 16), jnp.float32),
        pltpu.VMEM_SHARED((N_PAD,), jnp.float32),
        pltpu.VMEM_SHARED((N_PAD,), jnp.float32),
        pltpu.SemaphoreType.DMA,
        pltpu.SemaphoreType.DMA,
    ],
)


def kernel(x, edge_index, edge_attr,
           W1s, W1d, W1e, a1s, a1d, a1e, b1,
           W2s, W2d, W2e, a2s, a2d, a2e, b2,
           W3s, W3d, W3e, a3s, a3d, a3e, b3):
    n = x.shape[0]
    e = edge_index.shape[1]
    pad_n = ET_PAD - e - n

    x_pad = jnp.zeros((N_PAD, D), jnp.float32).at[:n].set(x)
    loop = jnp.arange(n, dtype=jnp.int32)
    pad_idx = (n + (jnp.arange(pad_n, dtype=jnp.int32) % (N_PAD - n)))
    src = jnp.concatenate([edge_index[0].astype(jnp.int32), loop, pad_idx])
    dst = jnp.concatenate([edge_index[1].astype(jnp.int32), loop, pad_idx])
    src3 = src.reshape(NW, CPT, CHUNK)
    dst3 = dst.reshape(NW, CPT, CHUNK)
    src2f = src.reshape(NW, EPW)

    st = _ea_stats(edge_attr.reshape(-1, 128))
    ea_mean = jnp.sum(st[0, :]) / e
    ea_max = jnp.max(st[1, :])
    ea_min = jnp.min(st[2, :])
    ea_full = jnp.concatenate([
        edge_attr.reshape(-1), jnp.full((n,), ea_mean, jnp.float32),
        jnp.zeros((pad_n,), jnp.float32)]).reshape(NW, CPT, CHUNK)

    W3s_p = jnp.zeros((D, D), jnp.float32).at[:, :1].set(W3s)
    W3d_p = jnp.zeros((D, D), jnp.float32).at[:, :1].set(W3d)
    a3s_p = jnp.zeros((D, 1), jnp.float32).at[:1].set(a3s[:, None])
    a3d_p = jnp.zeros((D, 1), jnp.float32).at[:1].set(a3d[:, None])

    layers = [
        (W1s, W1d, a1s.reshape(D, 1), a1d.reshape(D, 1), W1e, a1e, b1),
        (W2s, W2d, a2s.reshape(D, 1), a2d.reshape(D, 1), W2e, a2e, b2),
        (W3s_p, W3d_p, a3s_p, a3d_p, W3e, a3e, b3),
    ]

    op = None
    r = None
    for li, (ws, wd, avs, avd, we, ave, b) in enumerate(layers):
        if li == 0:
            xs, ss, sd, mx = _prep_from_x(x_pad, ws, wd, avs, avd)
        else:
            prev_b = layers[li - 1][6]
            xs, ss, sd, mx = _prep_from_partials(
                op, s2, prev_b, ws, wd, avs, avd)
        cl = jnp.sum(we[0] * ave)
        se_max = jnp.maximum(cl * ea_max, cl * ea_min)
        bb = mx[0, 0] + mx[1, 0] + se_max
        bb = jnp.where(bb >= 0.0, bb, 0.2 * bb)
        cb = jnp.stack([jnp.full((16,), cl, jnp.float32),
                        jnp.full((16,), bb, jnp.float32)])

        if li < 2:
            p, s2 = _pass1(src3, dst3, ea_full,
                           ss.reshape(N_PAD), sd.reshape(N_PAD), cb)
            op = _pass2(src.reshape(NW, P2, C2), dst.reshape(NW, P2, C2),
                        p.reshape(NW, P2, C2), xs)
        else:
            s2, o2 = _pass3(src2f, dst3, ea_full.reshape(NW, EPW),
                            ss.reshape(N_PAD), sd.reshape(N_PAD),
                            xs[:, 0], cb)
            out = _final(o2, s2, b)
    return out.reshape(N_PAD, 1)[:n]


# speedup vs baseline: 1.0740x; 1.0725x over previous
import functools

import jax
import jax.numpy as jnp
from jax import lax
from jax.experimental import pallas as pl
from jax.experimental.pallas import tpu as pltpu
from jax.experimental.pallas import tpu_sc as plsc

N_PAD = 10240
D = 128
CHUNK = 64
CPT = 162
EPW = CPT * CHUNK
NCORE = 2
NSUB = 16
NW = NCORE * NSUB
ET_PAD = NW * EPW
RPS = N_PAD // NSUB
BR = 256

_mesh = plsc.VectorSubcoreMesh(core_axis_name="c", subcore_axis_name="s",
                               num_cores=NCORE, num_subcores=NSUB)


def _ea_stats_body(ea_ref, o_ref):
    blk = ea_ref[...]
    o_ref[0, :] = jnp.sum(blk, axis=0)
    o_ref[1, :] = jnp.max(blk, axis=0)
    o_ref[2, :] = jnp.min(blk, axis=0)
    o_ref[3:, :] = jnp.zeros((5, 128), jnp.float32)


def _ea_stats(ea2d):
    return pl.pallas_call(
        _ea_stats_body,
        out_shape=jax.ShapeDtypeStruct((8, 128), jnp.float32),
    )(ea2d)


def _prep_x_body(h_ref, ws_ref, wd_ref, as_ref, ad_ref,
                 xs_ref, ss_ref, sd_ref, mx_ref):
    i = pl.program_id(0)
    h = h_ref[...]
    xs = jnp.dot(h, ws_ref[...], preferred_element_type=jnp.float32)
    xd = jnp.dot(h, wd_ref[...], preferred_element_type=jnp.float32)
    ss = jnp.dot(xs, as_ref[...], preferred_element_type=jnp.float32)
    sd = jnp.dot(xd, ad_ref[...], preferred_element_type=jnp.float32)
    xs_ref[...] = xs
    ss_ref[...] = ss
    sd_ref[...] = sd

    @pl.when(i == 0)
    def _():
        mx_ref[...] = jnp.full((8, 128), -jnp.inf)

    mx_ref[0, :] = jnp.maximum(mx_ref[0, :], jnp.max(ss))
    mx_ref[1, :] = jnp.maximum(mx_ref[1, :], jnp.max(sd))


def _prep_p_body(op_ref, s_ref, b_ref, ws_ref, wd_ref, as_ref, ad_ref,
                 xs_ref, ss_ref, sd_ref, mx_ref):
    i = pl.program_id(0)
    sb = s_ref[...]
    r = (1.0 / (sb[0] + sb[1] + 1e-16)).reshape(BR, 1)
    h = jnp.maximum((op_ref[0] + op_ref[1]) * r + b_ref[...], 0.0)
    xs = jnp.dot(h, ws_ref[...], preferred_element_type=jnp.float32)
    xd = jnp.dot(h, wd_ref[...], preferred_element_type=jnp.float32)
    ss = jnp.dot(xs, as_ref[...], preferred_element_type=jnp.float32)
    sd = jnp.dot(xd, ad_ref[...], preferred_element_type=jnp.float32)
    xs_ref[...] = xs
    ss_ref[...] = ss
    sd_ref[...] = sd

    @pl.when(i == 0)
    def _():
        mx_ref[...] = jnp.full((8, 128), -jnp.inf)

    mx_ref[0, :] = jnp.maximum(mx_ref[0, :], jnp.max(ss))
    mx_ref[1, :] = jnp.maximum(mx_ref[1, :], jnp.max(sd))


_PREP_OUT = (
    jax.ShapeDtypeStruct((N_PAD, D), jnp.float32),
    jax.ShapeDtypeStruct((N_PAD, 1), jnp.float32),
    jax.ShapeDtypeStruct((N_PAD, 1), jnp.float32),
    jax.ShapeDtypeStruct((8, 128), jnp.float32),
)
_PREP_OUT_SPECS = (
    pl.BlockSpec((BR, D), lambda i: (i, 0)),
    pl.BlockSpec((BR, 1), lambda i: (i, 0)),
    pl.BlockSpec((BR, 1), lambda i: (i, 0)),
    pl.BlockSpec((8, 128), lambda i: (0, 0)),
)
_W_SPECS = [
    pl.BlockSpec((D, D), lambda i: (0, 0)),
    pl.BlockSpec((D, D), lambda i: (0, 0)),
    pl.BlockSpec((D, 1), lambda i: (0, 0)),
    pl.BlockSpec((D, 1), lambda i: (0, 0)),
]


def _prep_from_x(h, ws, wd, a_s, a_d):
    return pl.pallas_call(
        _prep_x_body,
        grid=(N_PAD // BR,),
        in_specs=[pl.BlockSpec((BR, D), lambda i: (i, 0))] + _W_SPECS,
        out_specs=_PREP_OUT_SPECS,
        out_shape=_PREP_OUT,
    )(h, ws, wd, a_s, a_d)


def _prep_from_partials(op, s2, b, ws, wd, a_s, a_d):
    return pl.pallas_call(
        _prep_p_body,
        grid=(N_PAD // BR,),
        in_specs=[pl.BlockSpec((NCORE, BR, D), lambda i: (0, i, 0)),
                  pl.BlockSpec((NCORE, BR), lambda i: (0, i)),
                  pl.BlockSpec((1, D), lambda i: (0, 0))] + _W_SPECS,
        out_specs=_PREP_OUT_SPECS,
        out_shape=_PREP_OUT,
    )(op, s2, b.reshape(1, D), ws, wd, a_s, a_d)


def _final_body(o_ref, s_ref, b_ref, out_ref):
    r = 1.0 / (s_ref[0:1, :] + s_ref[1:2, :] + 1e-16)
    out_ref[...] = (o_ref[0:1, :] + o_ref[1:2, :]) * r + b_ref[0, 0]


def _final(o2, s2, b3):
    return pl.pallas_call(
        _final_body,
        in_specs=[pl.BlockSpec((NCORE, N_PAD), lambda: (0, 0)),
                  pl.BlockSpec((NCORE, N_PAD), lambda: (0, 0)),
                  pl.BlockSpec(memory_space=pltpu.SMEM)],
        out_shape=jax.ShapeDtypeStruct((1, N_PAD), jnp.float32),
    )(o2, s2, b3.reshape(1, 1))


def _zero_vec_ref(ref, nwords):
    z = jnp.zeros((16,), jnp.float32)

    def body(i, _):
        ref[pl.ds(i * 16, 16)] = z
        return 0

    lax.fori_loop(0, nwords // 16, body, 0)


def _zero_rows_ref(ref, nrows):
    z = jnp.zeros((16,), jnp.float32)

    def body(i, _):
        for j in range(D // 16):
            ref[i, pl.ds(j * 16, 16)] = z
        return 0

    lax.fori_loop(0, nrows, body, 0)


def _pass1_body(src_hbm, dst_hbm, ea_hbm, ss_hbm, sd_hbm, cb_hbm,
                p_hbm, s_hbm,
                ss_v, sd_v, src_v, dst_v, ea_v, p_v, stage_v, cb_v,
                s_sh, ssem):
    c = lax.axis_index("c")
    s = lax.axis_index("s")
    wid = s * NCORE + c

    _zero_vec_ref(stage_v, RPS)
    pltpu.sync_copy(stage_v, s_sh.at[pl.ds(s * RPS, RPS)])
    plsc.subcore_barrier()

    pltpu.sync_copy(ss_hbm, ss_v)
    pltpu.sync_copy(sd_hbm, sd_v)
    pltpu.sync_copy(cb_hbm, cb_v)
    pltpu.sync_copy(src_hbm.at[wid], src_v)
    pltpu.sync_copy(dst_hbm.at[wid], dst_v)
    pltpu.sync_copy(ea_hbm.at[wid], ea_v)

    cv = cb_v[0, :]
    bv = cb_v[1, :]

    def chunk(k, _):
        for j in range(CHUNK // 16):
            col = j * 16
            sidx = src_v[k, pl.ds(col, 16)]
            didx = dst_v[k, pl.ds(col, 16)]
            av = ea_v[k, pl.ds(col, 16)]
            e = (plsc.load_gather(ss_v, [sidx])
                 + plsc.load_gather(sd_v, [didx])
                 + cv * av)
            e = jnp.where(e >= 0.0, e, 0.2 * e)
            p_v[k, pl.ds(col, 16)] = jnp.exp(e - bv)
        return 0

    lax.fori_loop(0, CPT, chunk, 0)

    def scat(k, _):
        pltpu.async_copy(p_v.at[k], s_sh.at[dst_v.at[k]], ssem, add=True)
        return 0

    lax.fori_loop(0, CPT, scat, 0)

    pltpu.sync_copy(p_v, p_hbm.at[wid])

    def drain(k, _):
        pltpu.make_async_copy(p_v.at[k], s_sh.at[dst_v.at[k]], ssem).wait()
        return 0

    lax.fori_loop(0, CPT, drain, 0)
    plsc.subcore_barrier()

    pltpu.sync_copy(s_sh.at[pl.ds(s * RPS, RPS)], stage_v)
    pltpu.sync_copy(stage_v, s_hbm.at[c, pl.ds(s * RPS, RPS)])


_pass1 = pl.kernel(
    _pass1_body,
    out_type=(jax.ShapeDtypeStruct((NW, CPT, CHUNK), jnp.float32),
              jax.ShapeDtypeStruct((NCORE, N_PAD), jnp.float32)),
    mesh=_mesh,
    compiler_params=pltpu.CompilerParams(needs_layout_passes=False),
    scratch_types=[
        pltpu.VMEM((N_PAD,), jnp.float32),
        pltpu.VMEM((N_PAD,), jnp.float32),
        pltpu.VMEM((CPT, CHUNK), jnp.int32),
        pltpu.VMEM((CPT, CHUNK), jnp.int32),
        pltpu.VMEM((CPT, CHUNK), jnp.float32),
        pltpu.VMEM((CPT, CHUNK), jnp.float32),
        pltpu.VMEM((RPS,), jnp.float32),
        pltpu.VMEM((2, 16), jnp.float32),
        pltpu.VMEM_SHARED((N_PAD,), jnp.float32),
        pltpu.SemaphoreType.DMA,
    ],
)


def _scale_rows(rbuf, p_v, base, nrows):
    def sgrp(g, _):
        av16 = p_v[pl.ds(base + g * 16, 16)]
        for l in range(16):
            avec = jnp.full((16,), av16[l], jnp.float32)
            row = g * 16 + l
            for j in range(D // 16):
                col = j * 16
                rbuf[row, pl.ds(col, 16)] = rbuf[row, pl.ds(col, 16)] * avec
        return 0

    lax.fori_loop(0, nrows // 16, sgrp, 0)


def _pass2_body(src_hbm, dst_hbm, p_hbm, xs_hbm,
                o_hbm,
                src_v, p_v, dst_v, rows0_v, rows1_v,
                o_sh, g0, g1, d0, d1, sc0, sc1):
    c = lax.axis_index("c")
    s = lax.axis_index("s")
    wid = s * NCORE + c
    row0 = s * RPS

    _zero_rows_ref(rows0_v, CHUNK)
    for z in range(RPS // CHUNK):
        pltpu.sync_copy(rows0_v, o_sh.at[pl.ds(row0 + z * CHUNK, CHUNK)])
    plsc.subcore_barrier()

    pltpu.sync_copy(src_hbm.at[wid], src_v)
    pltpu.sync_copy(p_hbm.at[wid], p_v)

    pltpu.async_copy(xs_hbm.at[src_v.at[pl.ds(0, CHUNK)]], rows0_v, g0)
    pltpu.async_copy(dst_hbm.at[wid, 0], dst_v.at[0], d0)

    bufs = ((rows0_v, g0, d0, sc0), (rows1_v, g1, d1, sc1))

    def body(i, _):
        for par, (rbuf, gsem, dsem, ssem) in enumerate(bufs):
            k = i * 2 + par
            k1 = k + 1
            nbuf, ngsem, ndsem, nssem = bufs[1 - par]

            @pl.when(k1 < CPT)
            def _():
                @pl.when(k >= 1)
                def _():
                    pltpu.make_async_copy(
                        nbuf, o_sh.at[dst_v.at[1 - par]], nssem).wait()

                pltpu.async_copy(dst_hbm.at[wid, k1], dst_v.at[1 - par],
                                 ndsem)
                pltpu.async_copy(
                    xs_hbm.at[src_v.at[pl.ds(k1 * CHUNK, CHUNK)]],
                    nbuf, ngsem)

            pltpu.make_async_copy(
                xs_hbm.at[src_v.at[pl.ds(k * CHUNK, CHUNK)]],
                rbuf, gsem).wait()
            _scale_rows(rbuf, p_v, k * CHUNK, CHUNK)
            pltpu.make_async_copy(dst_hbm.at[wid, k], dst_v.at[par],
                                  dsem).wait()
            pltpu.async_copy(rbuf, o_sh.at[dst_v.at[par]], ssem, add=True)
        return 0

    lax.fori_loop(0, CPT // 2, body, 0)
    pltpu.make_async_copy(rows0_v, o_sh.at[dst_v.at[0]], sc0).wait()
    pltpu.make_async_copy(rows1_v, o_sh.at[dst_v.at[1]], sc1).wait()
    plsc.subcore_barrier()

    for z in range(RPS // CHUNK):
        pltpu.sync_copy(o_sh.at[pl.ds(row0 + z * CHUNK, CHUNK)], rows0_v)
        pltpu.sync_copy(rows0_v, o_hbm.at[c, pl.ds(row0 + z * CHUNK, CHUNK)])


_pass2 = pl.kernel(
    _pass2_body,
    out_type=jax.ShapeDtypeStruct((NCORE, N_PAD, D), jnp.float32),
    mesh=_mesh,
    compiler_params=pltpu.CompilerParams(needs_layout_passes=False),
    scratch_types=[
        pltpu.VMEM((EPW,), jnp.int32),
        pltpu.VMEM((EPW,), jnp.float32),
        pltpu.VMEM((2, CHUNK), jnp.int32),
        pltpu.VMEM((CHUNK, D), jnp.float32),
        pltpu.VMEM((CHUNK, D), jnp.float32),
        pltpu.VMEM_SHARED((N_PAD, D), jnp.float32),
        pltpu.SemaphoreType.DMA,
        pltpu.SemaphoreType.DMA,
        pltpu.SemaphoreType.DMA,
        pltpu.SemaphoreType.DMA,
        pltpu.SemaphoreType.DMA,
        pltpu.SemaphoreType.DMA,
    ],
)


def _pass3_body(src_hbm, dst_hbm, ea_hbm, ss_hbm, sd_hbm, xs_hbm, cb_hbm,
                s_hbm, o_hbm,
                ss_v, sd_v, xs_v, src_v, dst_v, ea_v, p_v, v_v,
                stage_v, cb_v,
                s_sh, o_sh, psem, vsem):
    c = lax.axis_index("c")
    s = lax.axis_index("s")
    wid = s * NCORE + c

    _zero_vec_ref(stage_v, RPS)
    pltpu.sync_copy(stage_v, s_sh.at[pl.ds(s * RPS, RPS)])
    pltpu.sync_copy(stage_v, o_sh.at[pl.ds(s * RPS, RPS)])
    plsc.subcore_barrier()

    pltpu.sync_copy(ss_hbm, ss_v)
    pltpu.sync_copy(sd_hbm, sd_v)
    pltpu.sync_copy(xs_hbm, xs_v)
    pltpu.sync_copy(cb_hbm, cb_v)
    pltpu.sync_copy(src_hbm.at[wid], src_v)
    pltpu.sync_copy(dst_hbm.at[wid], dst_v)
    pltpu.sync_copy(ea_hbm.at[wid], ea_v)

    cv = cb_v[0, :]
    bv = cb_v[1, :]

    def chunk(k, _):
        for j in range(CHUNK // 16):
            col = k * CHUNK + j * 16
            sidx = src_v[pl.ds(col, 16)]
            didx = dst_v[k, pl.ds(j * 16, 16)]
            av = ea_v[pl.ds(col, 16)]
            e = (plsc.load_gather(ss_v, [sidx])
                 + plsc.load_gather(sd_v, [didx])
                 + cv * av)
            e = jnp.where(e >= 0.0, e, 0.2 * e)
            p = jnp.exp(e - bv)
            p_v[pl.ds(col, 16)] = p
            v_v[pl.ds(col, 16)] = p * plsc.load_gather(xs_v, [sidx])
        return 0

    lax.fori_loop(0, CPT, chunk, 0)

    def scat(k, _):
        pltpu.async_copy(p_v.at[pl.ds(k * CHUNK, CHUNK)],
                         s_sh.at[dst_v.at[k]], psem, add=True)
        pltpu.async_copy(v_v.at[pl.ds(k * CHUNK, CHUNK)],
                         o_sh.at[dst_v.at[k]], vsem, add=True)
        return 0

    lax.fori_loop(0, CPT, scat, 0)

    def drain(k, _):
        pltpu.make_async_copy(p_v.at[pl.ds(k * CHUNK, CHUNK)],
                              s_sh.at[dst_v.at[k]], psem).wait()
        pltpu.make_async_copy(v_v.at[pl.ds(k * CHUNK, CHUNK)],
                              o_sh.at[dst_v.at[k]], vsem).wait()
        return 0

    lax.fori_loop(0, CPT, drain, 0)
    plsc.subcore_barrier()

    pltpu.sync_copy(s_sh.at[pl.ds(s * RPS, RPS)], stage_v)
    pltpu.sync_copy(stage_v, s_hbm.at[c, pl.ds(s * RPS, RPS)])
    pltpu.sync_copy(o_sh.at[pl.ds(s * RPS, RPS)], stage_v)
    pltpu.sync_copy(stage_v, o_hbm.at[c, pl.ds(s * RPS, RPS)])


_pass3 = pl.kernel(
    _pass3_body,
    out_type=(jax.ShapeDtypeStruct((NCORE, N_PAD), jnp.float32),
              jax.ShapeDtypeStruct((NCORE, N_PAD), jnp.float32)),
    mesh=_mesh,
    compiler_params=pltpu.CompilerParams(needs_layout_passes=False),
    scratch_types=[
        pltpu.VMEM((N_PAD,), jnp.float32),
        pltpu.VMEM((N_PAD,), jnp.float32),
        pltpu.VMEM((N_PAD,), jnp.float32),
        pltpu.VMEM((EPW,), jnp.int32),
        pltpu.VMEM((CPT, CHUNK), jnp.int32),
        pltpu.VMEM((EPW,), jnp.float32),
        pltpu.VMEM((EPW,), jnp.float32),
        pltpu.VMEM((EPW,), jnp.float32),
        pltpu.VMEM((RPS,), jnp.float32),
        pltpu.VMEM((2, 16), jnp.float32),
        pltpu.VMEM_SHARED((N_PAD,), jnp.float32),
        pltpu.VMEM_SHARED((N_PAD,), jnp.float32),
        pltpu.SemaphoreType.DMA,
        pltpu.SemaphoreType.DMA,
    ],
)


def kernel(x, edge_index, edge_attr,
           W1s, W1d, W1e, a1s, a1d, a1e, b1,
           W2s, W2d, W2e, a2s, a2d, a2e, b2,
           W3s, W3d, W3e, a3s, a3d, a3e, b3):
    n = x.shape[0]
    e = edge_index.shape[1]
    pad_n = ET_PAD - e - n

    x_pad = jnp.zeros((N_PAD, D), jnp.float32).at[:n].set(x)
    loop = jnp.arange(n, dtype=jnp.int32)
    pad_idx = (n + (jnp.arange(pad_n, dtype=jnp.int32) % (N_PAD - n)))
    src = jnp.concatenate([edge_index[0].astype(jnp.int32), loop, pad_idx])
    dst = jnp.concatenate([edge_index[1].astype(jnp.int32), loop, pad_idx])
    src3 = src.reshape(NW, CPT, CHUNK)
    dst3 = dst.reshape(NW, CPT, CHUNK)
    src2f = src.reshape(NW, EPW)

    st = _ea_stats(edge_attr.reshape(-1, 128))
    ea_mean = jnp.sum(st[0, :]) / e
    ea_max = jnp.max(st[1, :])
    ea_min = jnp.min(st[2, :])
    ea_full = jnp.concatenate([
        edge_attr.reshape(-1), jnp.full((n,), ea_mean, jnp.float32),
        jnp.zeros((pad_n,), jnp.float32)]).reshape(NW, CPT, CHUNK)

    W3s_p = jnp.zeros((D, D), jnp.float32).at[:, :1].set(W3s)
    W3d_p = jnp.zeros((D, D), jnp.float32).at[:, :1].set(W3d)
    a3s_p = jnp.zeros((D, 1), jnp.float32).at[:1].set(a3s[:, None])
    a3d_p = jnp.zeros((D, 1), jnp.float32).at[:1].set(a3d[:, None])

    layers = [
        (W1s, W1d, a1s.reshape(D, 1), a1d.reshape(D, 1), W1e, a1e, b1),
        (W2s, W2d, a2s.reshape(D, 1), a2d.reshape(D, 1), W2e, a2e, b2),
        (W3s_p, W3d_p, a3s_p, a3d_p, W3e, a3e, b3),
    ]

    op = None
    r = None
    for li, (ws, wd, avs, avd, we, ave, b) in enumerate(layers):
        if li == 0:
            xs, ss, sd, mx = _prep_from_x(x_pad, ws, wd, avs, avd)
        else:
            prev_b = layers[li - 1][6]
            xs, ss, sd, mx = _prep_from_partials(
                op, s2, prev_b, ws, wd, avs, avd)
        cl = jnp.sum(we[0] * ave)
        se_max = jnp.maximum(cl * ea_max, cl * ea_min)
        bb = mx[0, 0] + mx[1, 0] + se_max
        bb = jnp.where(bb >= 0.0, bb, 0.2 * bb)
        cb = jnp.stack([jnp.full((16,), cl, jnp.float32),
                        jnp.full((16,), bb, jnp.float32)])

        if li < 2:
            p, s2 = _pass1(src3, dst3, ea_full,
                           ss.reshape(N_PAD), sd.reshape(N_PAD), cb)
            op = _pass2(src2f, dst3, p.reshape(NW, EPW), xs)
        else:
            s2, o2 = _pass3(src2f, dst3, ea_full.reshape(NW, EPW),
                            ss.reshape(N_PAD), sd.reshape(N_PAD),
                            xs[:, 0], cb)
            out = _final(o2, s2, b)
    return out.reshape(N_PAD, 1)[:n]


# speedup vs baseline: 1.0948x; 1.0193x over previous
import functools

import jax
import jax.numpy as jnp
from jax import lax
from jax.experimental import pallas as pl
from jax.experimental.pallas import tpu as pltpu
from jax.experimental.pallas import tpu_sc as plsc

N_PAD = 10240
D = 128
CHUNK = 64
CPT = 162
EPW = CPT * CHUNK
NCORE = 2
NSUB = 16
NW = NCORE * NSUB
ET_PAD = NW * EPW
RPS = N_PAD // NSUB
BR = 256

_mesh = plsc.VectorSubcoreMesh(core_axis_name="c", subcore_axis_name="s",
                               num_cores=NCORE, num_subcores=NSUB)


def _ea_stats_body(ea_ref, o_ref):
    blk = ea_ref[...]
    o_ref[0, :] = jnp.sum(blk, axis=0)
    o_ref[1, :] = jnp.max(blk, axis=0)
    o_ref[2, :] = jnp.min(blk, axis=0)
    o_ref[3:, :] = jnp.zeros((5, 128), jnp.float32)


def _ea_stats(ea2d):
    return pl.pallas_call(
        _ea_stats_body,
        out_shape=jax.ShapeDtypeStruct((8, 128), jnp.float32),
    )(ea2d)


def _prep_x_body(h_ref, ws_ref, wd_ref, as_ref, ad_ref,
                 xs_ref, ss_ref, sd_ref, mx_ref):
    i = pl.program_id(0)
    h = h_ref[...]
    xs = jnp.dot(h, ws_ref[...], preferred_element_type=jnp.float32)
    xd = jnp.dot(h, wd_ref[...], preferred_element_type=jnp.float32)
    ss = jnp.dot(xs, as_ref[...], preferred_element_type=jnp.float32)
    sd = jnp.dot(xd, ad_ref[...], preferred_element_type=jnp.float32)
    xs_ref[...] = xs
    ss_ref[...] = ss
    sd_ref[...] = sd

    @pl.when(i == 0)
    def _():
        mx_ref[...] = jnp.full((8, 128), -jnp.inf)

    mx_ref[0, :] = jnp.maximum(mx_ref[0, :], jnp.max(ss))
    mx_ref[1, :] = jnp.maximum(mx_ref[1, :], jnp.max(sd))


def _prep_p_body(op_ref, s_ref, b_ref, ws_ref, wd_ref, as_ref, ad_ref,
                 xs_ref, ss_ref, sd_ref, mx_ref):
    i = pl.program_id(0)
    sb = s_ref[...]
    r = (1.0 / (sb[0] + sb[1] + 1e-16)).reshape(BR, 1)
    h = jnp.maximum((op_ref[0] + op_ref[1]) * r + b_ref[...], 0.0)
    xs = jnp.dot(h, ws_ref[...], preferred_element_type=jnp.float32)
    xd = jnp.dot(h, wd_ref[...], preferred_element_type=jnp.float32)
    ss = jnp.dot(xs, as_ref[...], preferred_element_type=jnp.float32)
    sd = jnp.dot(xd, ad_ref[...], preferred_element_type=jnp.float32)
    xs_ref[...] = xs
    ss_ref[...] = ss
    sd_ref[...] = sd

    @pl.when(i == 0)
    def _():
        mx_ref[...] = jnp.full((8, 128), -jnp.inf)

    mx_ref[0, :] = jnp.maximum(mx_ref[0, :], jnp.max(ss))
    mx_ref[1, :] = jnp.maximum(mx_ref[1, :], jnp.max(sd))


_PREP_OUT = (
    jax.ShapeDtypeStruct((N_PAD, D), jnp.float32),
    jax.ShapeDtypeStruct((N_PAD, 1), jnp.float32),
    jax.ShapeDtypeStruct((N_PAD, 1), jnp.float32),
    jax.ShapeDtypeStruct((8, 128), jnp.float32),
)
_PREP_OUT_SPECS = (
    pl.BlockSpec((BR, D), lambda i: (i, 0)),
    pl.BlockSpec((BR, 1), lambda i: (i, 0)),
    pl.BlockSpec((BR, 1), lambda i: (i, 0)),
    pl.BlockSpec((8, 128), lambda i: (0, 0)),
)
_W_SPECS = [
    pl.BlockSpec((D, D), lambda i: (0, 0)),
    pl.BlockSpec((D, D), lambda i: (0, 0)),
    pl.BlockSpec((D, 1), lambda i: (0, 0)),
    pl.BlockSpec((D, 1), lambda i: (0, 0)),
]


def _prep_from_x(h, ws, wd, a_s, a_d):
    return pl.pallas_call(
        _prep_x_body,
        grid=(N_PAD // BR,),
        in_specs=[pl.BlockSpec((BR, D), lambda i: (i, 0))] + _W_SPECS,
        out_specs=_PREP_OUT_SPECS,
        out_shape=_PREP_OUT,
    )(h, ws, wd, a_s, a_d)


def _prep_from_partials(op, s2, b, ws, wd, a_s, a_d):
    return pl.pallas_call(
        _prep_p_body,
        grid=(N_PAD // BR,),
        in_specs=[pl.BlockSpec((NCORE, BR, D), lambda i: (0, i, 0)),
                  pl.BlockSpec((NCORE, BR), lambda i: (0, i)),
                  pl.BlockSpec((1, D), lambda i: (0, 0))] + _W_SPECS,
        out_specs=_PREP_OUT_SPECS,
        out_shape=_PREP_OUT,
    )(op, s2, b.reshape(1, D), ws, wd, a_s, a_d)


def _final_body(o_ref, s_ref, b_ref, out_ref):
    r = 1.0 / (s_ref[0:1, :] + s_ref[1:2, :] + 1e-16)
    out_ref[...] = (o_ref[0:1, :] + o_ref[1:2, :]) * r + b_ref[0, 0]


def _final(o2, s2, b3):
    return pl.pallas_call(
        _final_body,
        in_specs=[pl.BlockSpec((NCORE, N_PAD), lambda: (0, 0)),
                  pl.BlockSpec((NCORE, N_PAD), lambda: (0, 0)),
                  pl.BlockSpec(memory_space=pltpu.SMEM)],
        out_shape=jax.ShapeDtypeStruct((1, N_PAD), jnp.float32),
    )(o2, s2, b3.reshape(1, 1))


def _zero_vec_ref(ref, nwords):
    z = jnp.zeros((16,), jnp.float32)

    def body(i, _):
        ref[pl.ds(i * 16, 16)] = z
        return 0

    lax.fori_loop(0, nwords // 16, body, 0)


def _zero_rows_ref(ref, nrows):
    z = jnp.zeros((16,), jnp.float32)

    def body(i, _):
        for j in range(D // 16):
            ref[i, pl.ds(j * 16, 16)] = z
        return 0

    lax.fori_loop(0, nrows, body, 0)


def _pass1_body(src_hbm, dst_hbm, ea_hbm, ss_hbm, sd_hbm, cb_hbm,
                p_hbm, s_hbm,
                ss_v, sd_v, src_v, dst_v, ea_v, p_v, stage_v, cb_v,
                s_sh, ssem):
    c = lax.axis_index("c")
    s = lax.axis_index("s")
    wid = s * NCORE + c

    _zero_vec_ref(stage_v, RPS)
    pltpu.sync_copy(stage_v, s_sh.at[pl.ds(s * RPS, RPS)])
    plsc.subcore_barrier()

    pltpu.async_copy(ss_hbm, ss_v, ssem)
    pltpu.async_copy(sd_hbm, sd_v, ssem)
    pltpu.async_copy(cb_hbm, cb_v, ssem)
    pltpu.async_copy(src_hbm.at[wid], src_v, ssem)
    pltpu.async_copy(dst_hbm.at[wid], dst_v, ssem)
    pltpu.async_copy(ea_hbm.at[wid], ea_v, ssem)
    pltpu.make_async_copy(ss_hbm, ss_v, ssem).wait()
    pltpu.make_async_copy(sd_hbm, sd_v, ssem).wait()
    pltpu.make_async_copy(cb_hbm, cb_v, ssem).wait()
    pltpu.make_async_copy(src_hbm.at[wid], src_v, ssem).wait()
    pltpu.make_async_copy(dst_hbm.at[wid], dst_v, ssem).wait()
    pltpu.make_async_copy(ea_hbm.at[wid], ea_v, ssem).wait()

    cv = cb_v[0, :]
    bv = cb_v[1, :]

    def chunk(k, _):
        for j in range(CHUNK // 16):
            col = j * 16
            sidx = src_v[k, pl.ds(col, 16)]
            didx = dst_v[k, pl.ds(col, 16)]
            av = ea_v[k, pl.ds(col, 16)]
            e = (plsc.load_gather(ss_v, [sidx])
                 + plsc.load_gather(sd_v, [didx])
                 + cv * av)
            e = jnp.where(e >= 0.0, e, 0.2 * e)
            p_v[k, pl.ds(col, 16)] = jnp.exp(e - bv)
        return 0

    lax.fori_loop(0, CPT, chunk, 0)

    def scat(k, _):
        pltpu.async_copy(p_v.at[k], s_sh.at[dst_v.at[k]], ssem, add=True)
        return 0

    lax.fori_loop(0, CPT, scat, 0)

    pltpu.sync_copy(p_v, p_hbm.at[wid])

    def drain(k, _):
        pltpu.make_async_copy(p_v.at[k], s_sh.at[dst_v.at[k]], ssem).wait()
        return 0

    lax.fori_loop(0, CPT, drain, 0)
    plsc.subcore_barrier()

    pltpu.sync_copy(s_sh.at[pl.ds(s * RPS, RPS)], stage_v)
    pltpu.sync_copy(stage_v, s_hbm.at[c, pl.ds(s * RPS, RPS)])


_pass1 = pl.kernel(
    _pass1_body,
    out_type=(jax.ShapeDtypeStruct((NW, CPT, CHUNK), jnp.float32),
              jax.ShapeDtypeStruct((NCORE, N_PAD), jnp.float32)),
    mesh=_mesh,
    compiler_params=pltpu.CompilerParams(needs_layout_passes=False),
    scratch_types=[
        pltpu.VMEM((N_PAD,), jnp.float32),
        pltpu.VMEM((N_PAD,), jnp.float32),
        pltpu.VMEM((CPT, CHUNK), jnp.int32),
        pltpu.VMEM((CPT, CHUNK), jnp.int32),
        pltpu.VMEM((CPT, CHUNK), jnp.float32),
        pltpu.VMEM((CPT, CHUNK), jnp.float32),
        pltpu.VMEM((RPS,), jnp.float32),
        pltpu.VMEM((2, 16), jnp.float32),
        pltpu.VMEM_SHARED((N_PAD,), jnp.float32),
        pltpu.SemaphoreType.DMA,
    ],
)


def _scale_rows(rbuf, p_v, base, nrows):
    def sgrp(g, _):
        av16 = p_v[pl.ds(base + g * 16, 16)]
        for l in range(16):
            avec = jnp.full((16,), av16[l], jnp.float32)
            row = g * 16 + l
            for j in range(D // 16):
                col = j * 16
                rbuf[row, pl.ds(col, 16)] = rbuf[row, pl.ds(col, 16)] * avec
        return 0

    lax.fori_loop(0, nrows // 16, sgrp, 0)


def _pass2_body(src_hbm, dst_hbm, p_hbm, xs_hbm,
                o_hbm,
                src_v, p_v, dst_v, rows0_v, rows1_v,
                o_sh, g0, g1, d0, d1, sc0, sc1):
    c = lax.axis_index("c")
    s = lax.axis_index("s")
    wid = s * NCORE + c
    row0 = s * RPS

    _zero_rows_ref(rows0_v, CHUNK)
    for z in range(RPS // CHUNK):
        pltpu.sync_copy(rows0_v, o_sh.at[pl.ds(row0 + z * CHUNK, CHUNK)])
    plsc.subcore_barrier()

    pltpu.async_copy(src_hbm.at[wid], src_v, d0)
    pltpu.async_copy(p_hbm.at[wid], p_v, d0)
    pltpu.make_async_copy(src_hbm.at[wid], src_v, d0).wait()
    pltpu.make_async_copy(p_hbm.at[wid], p_v, d0).wait()

    pltpu.async_copy(xs_hbm.at[src_v.at[pl.ds(0, CHUNK)]], rows0_v, g0)
    pltpu.async_copy(dst_hbm.at[wid, 0], dst_v.at[0], d0)

    bufs = ((rows0_v, g0, d0, sc0), (rows1_v, g1, d1, sc1))

    def body(i, _):
        for par, (rbuf, gsem, dsem, ssem) in enumerate(bufs):
            k = i * 2 + par
            k1 = k + 1
            nbuf, ngsem, ndsem, nssem = bufs[1 - par]

            @pl.when(k1 < CPT)
            def _():
                @pl.when(k >= 1)
                def _():
                    pltpu.make_async_copy(
                        nbuf, o_sh.at[dst_v.at[1 - par]], nssem).wait()

                pltpu.async_copy(dst_hbm.at[wid, k1], dst_v.at[1 - par],
                                 ndsem)
                pltpu.async_copy(
                    xs_hbm.at[src_v.at[pl.ds(k1 * CHUNK, CHUNK)]],
                    nbuf, ngsem)

            pltpu.make_async_copy(
                xs_hbm.at[src_v.at[pl.ds(k * CHUNK, CHUNK)]],
                rbuf, gsem).wait()
            _scale_rows(rbuf, p_v, k * CHUNK, CHUNK)
            pltpu.make_async_copy(dst_hbm.at[wid, k], dst_v.at[par],
                                  dsem).wait()
            pltpu.async_copy(rbuf, o_sh.at[dst_v.at[par]], ssem, add=True)
        return 0

    lax.fori_loop(0, CPT // 2, body, 0)
    pltpu.make_async_copy(rows0_v, o_sh.at[dst_v.at[0]], sc0).wait()
    pltpu.make_async_copy(rows1_v, o_sh.at[dst_v.at[1]], sc1).wait()
    plsc.subcore_barrier()

    for z in range(RPS // CHUNK):
        pltpu.sync_copy(o_sh.at[pl.ds(row0 + z * CHUNK, CHUNK)], rows0_v)
        pltpu.sync_copy(rows0_v, o_hbm.at[c, pl.ds(row0 + z * CHUNK, CHUNK)])


_pass2 = pl.kernel(
    _pass2_body,
    out_type=jax.ShapeDtypeStruct((NCORE, N_PAD, D), jnp.float32),
    mesh=_mesh,
    compiler_params=pltpu.CompilerParams(needs_layout_passes=False),
    scratch_types=[
        pltpu.VMEM((EPW,), jnp.int32),
        pltpu.VMEM((EPW,), jnp.float32),
        pltpu.VMEM((2, CHUNK), jnp.int32),
        pltpu.VMEM((CHUNK, D), jnp.float32),
        pltpu.VMEM((CHUNK, D), jnp.float32),
        pltpu.VMEM_SHARED((N_PAD, D), jnp.float32),
        pltpu.SemaphoreType.DMA,
        pltpu.SemaphoreType.DMA,
        pltpu.SemaphoreType.DMA,
        pltpu.SemaphoreType.DMA,
        pltpu.SemaphoreType.DMA,
        pltpu.SemaphoreType.DMA,
    ],
)


def _pass3_body(src_hbm, dst_hbm, ea_hbm, ss_hbm, sd_hbm, xs_hbm, cb_hbm,
                s_hbm, o_hbm,
                ss_v, sd_v, xs_v, src_v, dst_v, ea_v, p_v, v_v,
                stage_v, cb_v,
                s_sh, o_sh, psem, vsem):
    c = lax.axis_index("c")
    s = lax.axis_index("s")
    wid = s * NCORE + c

    _zero_vec_ref(stage_v, RPS)
    pltpu.sync_copy(stage_v, s_sh.at[pl.ds(s * RPS, RPS)])
    pltpu.sync_copy(stage_v, o_sh.at[pl.ds(s * RPS, RPS)])
    plsc.subcore_barrier()

    pltpu.async_copy(ss_hbm, ss_v, psem)
    pltpu.async_copy(sd_hbm, sd_v, psem)
    pltpu.async_copy(xs_hbm, xs_v, psem)
    pltpu.async_copy(cb_hbm, cb_v, psem)
    pltpu.async_copy(src_hbm.at[wid], src_v, psem)
    pltpu.async_copy(dst_hbm.at[wid], dst_v, psem)
    pltpu.async_copy(ea_hbm.at[wid], ea_v, psem)
    pltpu.make_async_copy(ss_hbm, ss_v, psem).wait()
    pltpu.make_async_copy(sd_hbm, sd_v, psem).wait()
    pltpu.make_async_copy(xs_hbm, xs_v, psem).wait()
    pltpu.make_async_copy(cb_hbm, cb_v, psem).wait()
    pltpu.make_async_copy(src_hbm.at[wid], src_v, psem).wait()
    pltpu.make_async_copy(dst_hbm.at[wid], dst_v, psem).wait()
    pltpu.make_async_copy(ea_hbm.at[wid], ea_v, psem).wait()

    cv = cb_v[0, :]
    bv = cb_v[1, :]

    def chunk(k, _):
        for j in range(CHUNK // 16):
            col = k * CHUNK + j * 16
            sidx = src_v[pl.ds(col, 16)]
            didx = dst_v[k, pl.ds(j * 16, 16)]
            av = ea_v[pl.ds(col, 16)]
            e = (plsc.load_gather(ss_v, [sidx])
                 + plsc.load_gather(sd_v, [didx])
                 + cv * av)
            e = jnp.where(e >= 0.0, e, 0.2 * e)
            p = jnp.exp(e - bv)
            p_v[pl.ds(col, 16)] = p
            v_v[pl.ds(col, 16)] = p * plsc.load_gather(xs_v, [sidx])
        return 0

    lax.fori_loop(0, CPT, chunk, 0)

    def scat(k, _):
        pltpu.async_copy(p_v.at[pl.ds(k * CHUNK, CHUNK)],
                         s_sh.at[dst_v.at[k]], psem, add=True)
        pltpu.async_copy(v_v.at[pl.ds(k * CHUNK, CHUNK)],
                         o_sh.at[dst_v.at[k]], vsem, add=True)
        return 0

    lax.fori_loop(0, CPT, scat, 0)

    def drain(k, _):
        pltpu.make_async_copy(p_v.at[pl.ds(k * CHUNK, CHUNK)],
                              s_sh.at[dst_v.at[k]], psem).wait()
        pltpu.make_async_copy(v_v.at[pl.ds(k * CHUNK, CHUNK)],
                              o_sh.at[dst_v.at[k]], vsem).wait()
        return 0

    lax.fori_loop(0, CPT, drain, 0)
    plsc.subcore_barrier()

    pltpu.sync_copy(s_sh.at[pl.ds(s * RPS, RPS)], stage_v)
    pltpu.sync_copy(stage_v, s_hbm.at[c, pl.ds(s * RPS, RPS)])
    pltpu.sync_copy(o_sh.at[pl.ds(s * RPS, RPS)], stage_v)
    pltpu.sync_copy(stage_v, o_hbm.at[c, pl.ds(s * RPS, RPS)])


_pass3 = pl.kernel(
    _pass3_body,
    out_type=(jax.ShapeDtypeStruct((NCORE, N_PAD), jnp.float32),
              jax.ShapeDtypeStruct((NCORE, N_PAD), jnp.float32)),
    mesh=_mesh,
    compiler_params=pltpu.CompilerParams(needs_layout_passes=False),
    scratch_types=[
        pltpu.VMEM((N_PAD,), jnp.float32),
        pltpu.VMEM((N_PAD,), jnp.float32),
        pltpu.VMEM((N_PAD,), jnp.float32),
        pltpu.VMEM((EPW,), jnp.int32),
        pltpu.VMEM((CPT, CHUNK), jnp.int32),
        pltpu.VMEM((EPW,), jnp.float32),
        pltpu.VMEM((EPW,), jnp.float32),
        pltpu.VMEM((EPW,), jnp.float32),
        pltpu.VMEM((RPS,), jnp.float32),
        pltpu.VMEM((2, 16), jnp.float32),
        pltpu.VMEM_SHARED((N_PAD,), jnp.float32),
        pltpu.VMEM_SHARED((N_PAD,), jnp.float32),
        pltpu.SemaphoreType.DMA,
        pltpu.SemaphoreType.DMA,
    ],
)


def kernel(x, edge_index, edge_attr,
           W1s, W1d, W1e, a1s, a1d, a1e, b1,
           W2s, W2d, W2e, a2s, a2d, a2e, b2,
           W3s, W3d, W3e, a3s, a3d, a3e, b3):
    n = x.shape[0]
    e = edge_index.shape[1]
    pad_n = ET_PAD - e - n

    x_pad = jnp.zeros((N_PAD, D), jnp.float32).at[:n].set(x)
    loop = jnp.arange(n, dtype=jnp.int32)
    pad_idx = (n + (jnp.arange(pad_n, dtype=jnp.int32) % (N_PAD - n)))
    src = jnp.concatenate([edge_index[0].astype(jnp.int32), loop, pad_idx])
    dst = jnp.concatenate([edge_index[1].astype(jnp.int32), loop, pad_idx])
    src3 = src.reshape(NW, CPT, CHUNK)
    dst3 = dst.reshape(NW, CPT, CHUNK)
    src2f = src.reshape(NW, EPW)

    st = _ea_stats(edge_attr.reshape(-1, 128))
    ea_mean = jnp.sum(st[0, :]) / e
    ea_max = jnp.max(st[1, :])
    ea_min = jnp.min(st[2, :])
    ea_full = jnp.concatenate([
        edge_attr.reshape(-1), jnp.full((n,), ea_mean, jnp.float32),
        jnp.zeros((pad_n,), jnp.float32)]).reshape(NW, CPT, CHUNK)

    W3s_p = jnp.zeros((D, D), jnp.float32).at[:, :1].set(W3s)
    W3d_p = jnp.zeros((D, D), jnp.float32).at[:, :1].set(W3d)
    a3s_p = jnp.zeros((D, 1), jnp.float32).at[:1].set(a3s[:, None])
    a3d_p = jnp.zeros((D, 1), jnp.float32).at[:1].set(a3d[:, None])

    layers = [
        (W1s, W1d, a1s.reshape(D, 1), a1d.reshape(D, 1), W1e, a1e, b1),
        (W2s, W2d, a2s.reshape(D, 1), a2d.reshape(D, 1), W2e, a2e, b2),
        (W3s_p, W3d_p, a3s_p, a3d_p, W3e, a3e, b3),
    ]

    op = None
    r = None
    for li, (ws, wd, avs, avd, we, ave, b) in enumerate(layers):
        if li == 0:
            xs, ss, sd, mx = _prep_from_x(x_pad, ws, wd, avs, avd)
        else:
            prev_b = layers[li - 1][6]
            xs, ss, sd, mx = _prep_from_partials(
                op, s2, prev_b, ws, wd, avs, avd)
        cl = jnp.sum(we[0] * ave)
        se_max = jnp.maximum(cl * ea_max, cl * ea_min)
        bb = mx[0, 0] + mx[1, 0] + se_max
        bb = jnp.where(bb >= 0.0, bb, 0.2 * bb)
        cb = jnp.stack([jnp.full((16,), cl, jnp.float32),
                        jnp.full((16,), bb, jnp.float32)])

        if li < 2:
            p, s2 = _pass1(src3, dst3, ea_full,
                           ss.reshape(N_PAD), sd.reshape(N_PAD), cb)
            op = _pass2(src2f, dst3, p.reshape(NW, EPW), xs)
        else:
            s2, o2 = _pass3(src2f, dst3, ea_full.reshape(NW, EPW),
                            ss.reshape(N_PAD), sd.reshape(N_PAD),
                            xs[:, 0], cb)
            out = _final(o2, s2, b)
    return out.reshape(N_PAD, 1)[:n]


# speedup vs baseline: 1.0951x; 1.0003x over previous
import functools

import jax
import jax.numpy as jnp
from jax import lax
from jax.experimental import pallas as pl
from jax.experimental.pallas import tpu as pltpu
from jax.experimental.pallas import tpu_sc as plsc

N_PAD = 10240
D = 128
CHUNK = 64
CPT = 162
EPW = CPT * CHUNK
NCORE = 2
NSUB = 16
NW = NCORE * NSUB
ET_PAD = NW * EPW
RPS = N_PAD // NSUB
BR = 256

_mesh = plsc.VectorSubcoreMesh(core_axis_name="c", subcore_axis_name="s",
                               num_cores=NCORE, num_subcores=NSUB)


def _prep_x_body(h_ref, ea_ref, ws_ref, wd_ref, as_ref, ad_ref,
                 xs_ref, ss_ref, sd_ref, mx_ref):
    i = pl.program_id(0)
    h = h_ref[...]
    xs = jnp.dot(h, ws_ref[...], preferred_element_type=jnp.float32)
    xd = jnp.dot(h, wd_ref[...], preferred_element_type=jnp.float32)
    ss = jnp.dot(xs, as_ref[...], preferred_element_type=jnp.float32)
    sd = jnp.dot(xd, ad_ref[...], preferred_element_type=jnp.float32)
    xs_ref[...] = xs
    ss_ref[...] = ss
    sd_ref[...] = sd

    @pl.when(i == 0)
    def _():
        mx_ref[...] = jnp.full((8, 128), -jnp.inf)
        mx_ref[3, :] = jnp.full((128,), jnp.inf)
        mx_ref[4, :] = jnp.zeros((128,), jnp.float32)

    mx_ref[0, :] = jnp.maximum(mx_ref[0, :], jnp.max(ss))
    mx_ref[1, :] = jnp.maximum(mx_ref[1, :], jnp.max(sd))
    blk = ea_ref[...]
    mx_ref[2, :] = jnp.maximum(mx_ref[2, :], jnp.max(blk, axis=0))
    mx_ref[3, :] = jnp.minimum(mx_ref[3, :], jnp.min(blk, axis=0))
    mx_ref[4, :] = mx_ref[4, :] + jnp.sum(blk, axis=0)


def _prep_p_body(op_ref, s_ref, b_ref, ws_ref, wd_ref, as_ref, ad_ref,
                 xs_ref, ss_ref, sd_ref, mx_ref):
    i = pl.program_id(0)
    sb = s_ref[...]
    r = (1.0 / (sb[0] + sb[1] + 1e-16)).reshape(BR, 1)
    h = jnp.maximum((op_ref[0] + op_ref[1]) * r + b_ref[...], 0.0)
    xs = jnp.dot(h, ws_ref[...], preferred_element_type=jnp.float32)
    xd = jnp.dot(h, wd_ref[...], preferred_element_type=jnp.float32)
    ss = jnp.dot(xs, as_ref[...], preferred_element_type=jnp.float32)
    sd = jnp.dot(xd, ad_ref[...], preferred_element_type=jnp.float32)
    xs_ref[...] = xs
    ss_ref[...] = ss
    sd_ref[...] = sd

    @pl.when(i == 0)
    def _():
        mx_ref[...] = jnp.full((8, 128), -jnp.inf)

    mx_ref[0, :] = jnp.maximum(mx_ref[0, :], jnp.max(ss))
    mx_ref[1, :] = jnp.maximum(mx_ref[1, :], jnp.max(sd))


_PREP_OUT = (
    jax.ShapeDtypeStruct((N_PAD, D), jnp.float32),
    jax.ShapeDtypeStruct((N_PAD, 1), jnp.float32),
    jax.ShapeDtypeStruct((N_PAD, 1), jnp.float32),
    jax.ShapeDtypeStruct((8, 128), jnp.float32),
)
_PREP_OUT_SPECS = (
    pl.BlockSpec((BR, D), lambda i: (i, 0)),
    pl.BlockSpec((BR, 1), lambda i: (i, 0)),
    pl.BlockSpec((BR, 1), lambda i: (i, 0)),
    pl.BlockSpec((8, 128), lambda i: (0, 0)),
)
_W_SPECS = [
    pl.BlockSpec((D, D), lambda i: (0, 0)),
    pl.BlockSpec((D, D), lambda i: (0, 0)),
    pl.BlockSpec((D, 1), lambda i: (0, 0)),
    pl.BlockSpec((D, 1), lambda i: (0, 0)),
]


def _prep_from_x(h, ea2d, ws, wd, a_s, a_d):
    return pl.pallas_call(
        _prep_x_body,
        grid=(N_PAD // BR,),
        in_specs=[pl.BlockSpec((BR, D), lambda i: (i, 0)),
                  pl.BlockSpec((64, 128), lambda i: (i, 0))] + _W_SPECS,
        out_specs=_PREP_OUT_SPECS,
        out_shape=_PREP_OUT,
    )(h, ea2d, ws, wd, a_s, a_d)


def _prep_from_partials(op, s2, b, ws, wd, a_s, a_d):
    return pl.pallas_call(
        _prep_p_body,
        grid=(N_PAD // BR,),
        in_specs=[pl.BlockSpec((NCORE, BR, D), lambda i: (0, i, 0)),
                  pl.BlockSpec((NCORE, BR), lambda i: (0, i)),
                  pl.BlockSpec((1, D), lambda i: (0, 0))] + _W_SPECS,
        out_specs=_PREP_OUT_SPECS,
        out_shape=_PREP_OUT,
    )(op, s2, b.reshape(1, D), ws, wd, a_s, a_d)


def _final_body(o_ref, s_ref, b_ref, out_ref):
    r = 1.0 / (s_ref[0:1, :] + s_ref[1:2, :] + 1e-16)
    out_ref[...] = (o_ref[0:1, :] + o_ref[1:2, :]) * r + b_ref[0, 0]


def _final(o2, s2, b3):
    return pl.pallas_call(
        _final_body,
        in_specs=[pl.BlockSpec((NCORE, N_PAD), lambda: (0, 0)),
                  pl.BlockSpec((NCORE, N_PAD), lambda: (0, 0)),
                  pl.BlockSpec(memory_space=pltpu.SMEM)],
        out_shape=jax.ShapeDtypeStruct((1, N_PAD), jnp.float32),
    )(o2, s2, b3.reshape(1, 1))


def _zero_vec_ref(ref, nwords):
    z = jnp.zeros((16,), jnp.float32)

    def body(i, _):
        ref[pl.ds(i * 16, 16)] = z
        return 0

    lax.fori_loop(0, nwords // 16, body, 0)


def _zero_rows_ref(ref, nrows):
    z = jnp.zeros((16,), jnp.float32)

    def body(i, _):
        for j in range(D // 16):
            ref[i, pl.ds(j * 16, 16)] = z
        return 0

    lax.fori_loop(0, nrows, body, 0)


def _pass1_body(src_hbm, dst_hbm, ea_hbm, ss_hbm, sd_hbm, cb_hbm,
                p_hbm, s_hbm,
                ss_v, sd_v, src_v, dst_v, ea_v, p_v, stage_v, cb_v,
                s_sh, ssem):
    c = lax.axis_index("c")
    s = lax.axis_index("s")
    wid = s * NCORE + c

    _zero_vec_ref(stage_v, RPS)
    pltpu.sync_copy(stage_v, s_sh.at[pl.ds(s * RPS, RPS)])
    plsc.subcore_barrier()

    pltpu.async_copy(ss_hbm, ss_v, ssem)
    pltpu.async_copy(sd_hbm, sd_v, ssem)
    pltpu.async_copy(cb_hbm, cb_v, ssem)
    pltpu.async_copy(src_hbm.at[wid], src_v, ssem)
    pltpu.async_copy(dst_hbm.at[wid], dst_v, ssem)
    pltpu.async_copy(ea_hbm.at[wid], ea_v, ssem)
    pltpu.make_async_copy(ss_hbm, ss_v, ssem).wait()
    pltpu.make_async_copy(sd_hbm, sd_v, ssem).wait()
    pltpu.make_async_copy(cb_hbm, cb_v, ssem).wait()
    pltpu.make_async_copy(src_hbm.at[wid], src_v, ssem).wait()
    pltpu.make_async_copy(dst_hbm.at[wid], dst_v, ssem).wait()
    pltpu.make_async_copy(ea_hbm.at[wid], ea_v, ssem).wait()

    cv = cb_v[0, :]
    bv = cb_v[1, :]

    def chunk(k, _):
        for j in range(CHUNK // 16):
            col = j * 16
            sidx = src_v[k, pl.ds(col, 16)]
            didx = dst_v[k, pl.ds(col, 16)]
            av = ea_v[k, pl.ds(col, 16)]
            e = (plsc.load_gather(ss_v, [sidx])
                 + plsc.load_gather(sd_v, [didx])
                 + cv * av)
            e = jnp.where(e >= 0.0, e, 0.2 * e)
            p_v[k, pl.ds(col, 16)] = jnp.exp(e - bv)
        return 0

    lax.fori_loop(0, CPT, chunk, 0)

    def scat(k, _):
        pltpu.async_copy(p_v.at[k], s_sh.at[dst_v.at[k]], ssem, add=True)
        return 0

    lax.fori_loop(0, CPT, scat, 0)

    pltpu.sync_copy(p_v, p_hbm.at[wid])

    def drain(k, _):
        pltpu.make_async_copy(p_v.at[k], s_sh.at[dst_v.at[k]], ssem).wait()
        return 0

    lax.fori_loop(0, CPT, drain, 0)
    plsc.subcore_barrier()

    pltpu.sync_copy(s_sh.at[pl.ds(s * RPS, RPS)], stage_v)
    pltpu.sync_copy(stage_v, s_hbm.at[c, pl.ds(s * RPS, RPS)])


_pass1 = pl.kernel(
    _pass1_body,
    out_type=(jax.ShapeDtypeStruct((NW, CPT, CHUNK), jnp.float32),
              jax.ShapeDtypeStruct((NCORE, N_PAD), jnp.float32)),
    mesh=_mesh,
    compiler_params=pltpu.CompilerParams(needs_layout_passes=False),
    scratch_types=[
        pltpu.VMEM((N_PAD,), jnp.float32),
        pltpu.VMEM((N_PAD,), jnp.float32),
        pltpu.VMEM((CPT, CHUNK), jnp.int32),
        pltpu.VMEM((CPT, CHUNK), jnp.int32),
        pltpu.VMEM((CPT, CHUNK), jnp.float32),
        pltpu.VMEM((CPT, CHUNK), jnp.float32),
        pltpu.VMEM((RPS,), jnp.float32),
        pltpu.VMEM((2, 16), jnp.float32),
        pltpu.VMEM_SHARED((N_PAD,), jnp.float32),
        pltpu.SemaphoreType.DMA,
    ],
)


def _scale_rows(rbuf, p_v, base, nrows):
    def sgrp(g, _):
        av16 = p_v[pl.ds(base + g * 16, 16)]
        for l in range(16):
            avec = jnp.full((16,), av16[l], jnp.float32)
            row = g * 16 + l
            for j in range(D // 16):
                col = j * 16
                rbuf[row, pl.ds(col, 16)] = rbuf[row, pl.ds(col, 16)] * avec
        return 0

    lax.fori_loop(0, nrows // 16, sgrp, 0)


def _pass2_body(src_hbm, dst_hbm, p_hbm, xs_hbm,
                o_hbm,
                src_v, p_v, dst_v, rows0_v, rows1_v,
                o_sh, g0, g1, d0, d1, sc0, sc1):
    c = lax.axis_index("c")
    s = lax.axis_index("s")
    wid = s * NCORE + c
    row0 = s * RPS

    _zero_rows_ref(rows0_v, CHUNK)
    for z in range(RPS // CHUNK):
        pltpu.sync_copy(rows0_v, o_sh.at[pl.ds(row0 + z * CHUNK, CHUNK)])
    plsc.subcore_barrier()

    pltpu.async_copy(src_hbm.at[wid], src_v, d0)
    pltpu.async_copy(p_hbm.at[wid], p_v, d0)
    pltpu.make_async_copy(src_hbm.at[wid], src_v, d0).wait()
    pltpu.make_async_copy(p_hbm.at[wid], p_v, d0).wait()

    pltpu.async_copy(xs_hbm.at[src_v.at[pl.ds(0, CHUNK)]], rows0_v, g0)
    pltpu.async_copy(dst_hbm.at[wid, 0], dst_v.at[0], d0)

    bufs = ((rows0_v, g0, d0, sc0), (rows1_v, g1, d1, sc1))

    def body(i, _):
        for par, (rbuf, gsem, dsem, ssem) in enumerate(bufs):
            k = i * 2 + par
            k1 = k + 1
            nbuf, ngsem, ndsem, nssem = bufs[1 - par]

            @pl.when(k1 < CPT)
            def _():
                @pl.when(k >= 1)
                def _():
                    pltpu.make_async_copy(
                        nbuf, o_sh.at[dst_v.at[1 - par]], nssem).wait()

                pltpu.async_copy(dst_hbm.at[wid, k1], dst_v.at[1 - par],
                                 ndsem)
                pltpu.async_copy(
                    xs_hbm.at[src_v.at[pl.ds(k1 * CHUNK, CHUNK)]],
                    nbuf, ngsem)

            pltpu.make_async_copy(
                xs_hbm.at[src_v.at[pl.ds(k * CHUNK, CHUNK)]],
                rbuf, gsem).wait()
            _scale_rows(rbuf, p_v, k * CHUNK, CHUNK)
            pltpu.make_async_copy(dst_hbm.at[wid, k], dst_v.at[par],
                                  dsem).wait()
            pltpu.async_copy(rbuf, o_sh.at[dst_v.at[par]], ssem, add=True)
        return 0

    lax.fori_loop(0, CPT // 2, body, 0)
    pltpu.make_async_copy(rows0_v, o_sh.at[dst_v.at[0]], sc0).wait()
    pltpu.make_async_copy(rows1_v, o_sh.at[dst_v.at[1]], sc1).wait()
    plsc.subcore_barrier()

    for z in range(RPS // CHUNK):
        pltpu.sync_copy(o_sh.at[pl.ds(row0 + z * CHUNK, CHUNK)], rows0_v)
        pltpu.sync_copy(rows0_v, o_hbm.at[c, pl.ds(row0 + z * CHUNK, CHUNK)])


_pass2 = pl.kernel(
    _pass2_body,
    out_type=jax.ShapeDtypeStruct((NCORE, N_PAD, D), jnp.float32),
    mesh=_mesh,
    compiler_params=pltpu.CompilerParams(needs_layout_passes=False),
    scratch_types=[
        pltpu.VMEM((EPW,), jnp.int32),
        pltpu.VMEM((EPW,), jnp.float32),
        pltpu.VMEM((2, CHUNK), jnp.int32),
        pltpu.VMEM((CHUNK, D), jnp.float32),
        pltpu.VMEM((CHUNK, D), jnp.float32),
        pltpu.VMEM_SHARED((N_PAD, D), jnp.float32),
        pltpu.SemaphoreType.DMA,
        pltpu.SemaphoreType.DMA,
        pltpu.SemaphoreType.DMA,
        pltpu.SemaphoreType.DMA,
        pltpu.SemaphoreType.DMA,
        pltpu.SemaphoreType.DMA,
    ],
)


def _pass3_body(src_hbm, dst_hbm, ea_hbm, ss_hbm, sd_hbm, xs_hbm, cb_hbm,
                s_hbm, o_hbm,
                ss_v, sd_v, xs_v, src_v, dst_v, ea_v, p_v, v_v,
                stage_v, cb_v,
                s_sh, o_sh, psem, vsem):
    c = lax.axis_index("c")
    s = lax.axis_index("s")
    wid = s * NCORE + c

    _zero_vec_ref(stage_v, RPS)
    pltpu.sync_copy(stage_v, s_sh.at[pl.ds(s * RPS, RPS)])
    pltpu.sync_copy(stage_v, o_sh.at[pl.ds(s * RPS, RPS)])
    plsc.subcore_barrier()

    pltpu.async_copy(ss_hbm, ss_v, psem)
    pltpu.async_copy(sd_hbm, sd_v, psem)
    pltpu.async_copy(xs_hbm, xs_v, psem)
    pltpu.async_copy(cb_hbm, cb_v, psem)
    pltpu.async_copy(src_hbm.at[wid], src_v, psem)
    pltpu.async_copy(dst_hbm.at[wid], dst_v, psem)
    pltpu.async_copy(ea_hbm.at[wid], ea_v, psem)
    pltpu.make_async_copy(ss_hbm, ss_v, psem).wait()
    pltpu.make_async_copy(sd_hbm, sd_v, psem).wait()
    pltpu.make_async_copy(xs_hbm, xs_v, psem).wait()
    pltpu.make_async_copy(cb_hbm, cb_v, psem).wait()
    pltpu.make_async_copy(src_hbm.at[wid], src_v, psem).wait()
    pltpu.make_async_copy(dst_hbm.at[wid], dst_v, psem).wait()
    pltpu.make_async_copy(ea_hbm.at[wid], ea_v, psem).wait()

    cv = cb_v[0, :]
    bv = cb_v[1, :]

    def chunk(k, _):
        for j in range(CHUNK // 16):
            col = k * CHUNK + j * 16
            sidx = src_v[pl.ds(col, 16)]
            didx = dst_v[k, pl.ds(j * 16, 16)]
            av = ea_v[pl.ds(col, 16)]
            e = (plsc.load_gather(ss_v, [sidx])
                 + plsc.load_gather(sd_v, [didx])
                 + cv * av)
            e = jnp.where(e >= 0.0, e, 0.2 * e)
            p = jnp.exp(e - bv)
            p_v[pl.ds(col, 16)] = p
            v_v[pl.ds(col, 16)] = p * plsc.load_gather(xs_v, [sidx])
        return 0

    lax.fori_loop(0, CPT, chunk, 0)

    def scat(k, _):
        pltpu.async_copy(p_v.at[pl.ds(k * CHUNK, CHUNK)],
                         s_sh.at[dst_v.at[k]], psem, add=True)
        pltpu.async_copy(v_v.at[pl.ds(k * CHUNK, CHUNK)],
                         o_sh.at[dst_v.at[k]], vsem, add=True)
        return 0

    lax.fori_loop(0, CPT, scat, 0)

    def drain(k, _):
        pltpu.make_async_copy(p_v.at[pl.ds(k * CHUNK, CHUNK)],
                              s_sh.at[dst_v.at[k]], psem).wait()
        pltpu.make_async_copy(v_v.at[pl.ds(k * CHUNK, CHUNK)],
                              o_sh.at[dst_v.at[k]], vsem).wait()
        return 0

    lax.fori_loop(0, CPT, drain, 0)
    plsc.subcore_barrier()

    pltpu.sync_copy(s_sh.at[pl.ds(s * RPS, RPS)], stage_v)
    pltpu.sync_copy(stage_v, s_hbm.at[c, pl.ds(s * RPS, RPS)])
    pltpu.sync_copy(o_sh.at[pl.ds(s * RPS, RPS)], stage_v)
    pltpu.sync_copy(stage_v, o_hbm.at[c, pl.ds(s * RPS, RPS)])


_pass3 = pl.kernel(
    _pass3_body,
    out_type=(jax.ShapeDtypeStruct((NCORE, N_PAD), jnp.float32),
              jax.ShapeDtypeStruct((NCORE, N_PAD), jnp.float32)),
    mesh=_mesh,
    compiler_params=pltpu.CompilerParams(needs_layout_passes=False),
    scratch_types=[
        pltpu.VMEM((N_PAD,), jnp.float32),
        pltpu.VMEM((N_PAD,), jnp.float32),
        pltpu.VMEM((N_PAD,), jnp.float32),
        pltpu.VMEM((EPW,), jnp.int32),
        pltpu.VMEM((CPT, CHUNK), jnp.int32),
        pltpu.VMEM((EPW,), jnp.float32),
        pltpu.VMEM((EPW,), jnp.float32),
        pltpu.VMEM((EPW,), jnp.float32),
        pltpu.VMEM((RPS,), jnp.float32),
        pltpu.VMEM((2, 16), jnp.float32),
        pltpu.VMEM_SHARED((N_PAD,), jnp.float32),
        pltpu.VMEM_SHARED((N_PAD,), jnp.float32),
        pltpu.SemaphoreType.DMA,
        pltpu.SemaphoreType.DMA,
    ],
)


def kernel(x, edge_index, edge_attr,
           W1s, W1d, W1e, a1s, a1d, a1e, b1,
           W2s, W2d, W2e, a2s, a2d, a2e, b2,
           W3s, W3d, W3e, a3s, a3d, a3e, b3):
    n = x.shape[0]
    e = edge_index.shape[1]
    pad_n = ET_PAD - e - n

    x_pad = jnp.zeros((N_PAD, D), jnp.float32).at[:n].set(x)
    loop = jnp.arange(n, dtype=jnp.int32)
    pad_idx = (n + (jnp.arange(pad_n, dtype=jnp.int32) % (N_PAD - n)))
    src = jnp.concatenate([edge_index[0].astype(jnp.int32), loop, pad_idx])
    dst = jnp.concatenate([edge_index[1].astype(jnp.int32), loop, pad_idx])
    src3 = src.reshape(NW, CPT, CHUNK)
    dst3 = dst.reshape(NW, CPT, CHUNK)
    src2f = src.reshape(NW, EPW)

    ea_flat = edge_attr.reshape(-1)
    ea_pad2d = jnp.zeros((2560 * 128,), jnp.float32).at[:e].set(
        ea_flat).reshape(2560, 128)

    W3s_p = jnp.zeros((D, D), jnp.float32).at[:, :1].set(W3s)
    W3d_p = jnp.zeros((D, D), jnp.float32).at[:, :1].set(W3d)
    a3s_p = jnp.zeros((D, 1), jnp.float32).at[:1].set(a3s[:, None])
    a3d_p = jnp.zeros((D, 1), jnp.float32).at[:1].set(a3d[:, None])

    layers = [
        (W1s, W1d, a1s.reshape(D, 1), a1d.reshape(D, 1), W1e, a1e, b1),
        (W2s, W2d, a2s.reshape(D, 1), a2d.reshape(D, 1), W2e, a2e, b2),
        (W3s_p, W3d_p, a3s_p, a3d_p, W3e, a3e, b3),
    ]

    op = None
    ea_full = None
    for li, (ws, wd, avs, avd, we, ave, b) in enumerate(layers):
        if li == 0:
            xs, ss, sd, mx = _prep_from_x(x_pad, ea_pad2d, ws, wd, avs, avd)
            ea_mean = jnp.sum(mx[4, :]) / e
            ea_max = jnp.max(mx[2, :])
            ea_min = jnp.min(mx[3, :])
            ea_full = jnp.concatenate([
                ea_flat, jnp.full((n,), ea_mean, jnp.float32),
                jnp.zeros((pad_n,), jnp.float32)]).reshape(NW, CPT, CHUNK)
        else:
            prev_b = layers[li - 1][6]
            xs, ss, sd, mx = _prep_from_partials(
                op, s2, prev_b, ws, wd, avs, avd)
        cl = jnp.sum(we[0] * ave)
        se_max = jnp.maximum(cl * ea_max, cl * ea_min)
        bb = mx[0, 0] + mx[1, 0] + se_max
        bb = jnp.where(bb >= 0.0, bb, 0.2 * bb)
        cb = jnp.stack([jnp.full((16,), cl, jnp.float32),
                        jnp.full((16,), bb, jnp.float32)])

        if li < 2:
            p, s2 = _pass1(src3, dst3, ea_full,
                           ss.reshape(N_PAD), sd.reshape(N_PAD), cb)
            op = _pass2(src2f, dst3, p.reshape(NW, EPW), xs)
        else:
            s2, o2 = _pass3(src2f, dst3, ea_full.reshape(NW, EPW),
                            ss.reshape(N_PAD), sd.reshape(N_PAD),
                            xs[:, 0], cb)
            out = _final(o2, s2, b)
    return out.reshape(N_PAD, 1)[:n]
